# baseline (speedup 1.0000x reference)
import functools

import jax
import jax.numpy as jnp
from jax import lax
from jax.experimental import pallas as pl
from jax.experimental.pallas import tpu as pltpu
from jax.experimental.pallas import tpu_sc as plsc

N = 10000
E = 320000
NF = 128
NEF = 16
MSG = 128
HID = 300
NH = 128
NP = 2
NG = 64

NC = 2
NS = 16
NW = NC * NS
PER_W = E // NW
C = 80
NCHUNK = PER_W // C
NAGG = 10240
ROWS_PER_TILE = NAGG // NS


NPK = NF // 2


def _gather_body(x_hbm, src_hbm, dst_hbm, xj_hbm, xi_hbm,
                 idx_a, rows_a, idx_b, rows_b, sem_a, sem_b):
    c = lax.axis_index("c")
    s = lax.axis_index("s")
    base = (c * NS + s) * PER_W

    @pl.loop(0, NCHUNK)
    def _(j):
        off = base + j * C
        pltpu.sync_copy(src_hbm.at[pl.ds(off, C)], idx_a)
        cp_a = pltpu.async_copy(x_hbm.at[idx_a], rows_a, sem_a)
        pltpu.sync_copy(dst_hbm.at[pl.ds(off, C)], idx_b)
        cp_b = pltpu.async_copy(x_hbm.at[idx_b], rows_b, sem_b)
        cp_a.wait()
        pltpu.sync_copy(rows_a, xj_hbm.at[pl.ds(off, C)])
        cp_b.wait()
        pltpu.sync_copy(rows_b, xi_hbm.at[pl.ds(off, C)])


def _sc_gather(xp, src, dst):
    mesh = plsc.VectorSubcoreMesh(core_axis_name="c", subcore_axis_name="s")
    f = pl.kernel(
        _gather_body,
        out_type=(
            jax.ShapeDtypeStruct((E, NPK), jnp.int32),
            jax.ShapeDtypeStruct((E, NPK), jnp.int32),
        ),
        mesh=mesh,
        scratch_types=[
            pltpu.VMEM((C,), jnp.int32),
            pltpu.VMEM((C, NPK), jnp.int32),
            pltpu.VMEM((C,), jnp.int32),
            pltpu.VMEM((C, NPK), jnp.int32),
            pltpu.SemaphoreType.DMA,
            pltpu.SemaphoreType.DMA,
        ],
        compiler_params=pltpu.CompilerParams(use_tc_tiling_on_sc=False),
    )
    return f(xp, src, dst)


def _pack_x(x):
    bits = lax.bitcast_convert_type(x.astype(jnp.bfloat16), jnp.uint16)
    lo = bits[:, :NPK].astype(jnp.uint32)
    hi = bits[:, NPK:].astype(jnp.uint32)
    return lax.bitcast_convert_type(lo | (hi << 16), jnp.int32)


def _unpack(w):
    lo = lax.bitcast_convert_type(w << 16, jnp.float32)
    hi = lax.bitcast_convert_type(w & jnp.int32(-65536), jnp.float32)
    return lo, hi


def _scatter_body(msg_hbm, dst_hbm, z_hbm, out_hbm, idx_v, rows_v, acc_sh):
    c = lax.axis_index("c")
    s = lax.axis_index("s")
    pltpu.sync_copy(z_hbm, acc_sh.at[pl.ds(s * ROWS_PER_TILE, ROWS_PER_TILE)])
    plsc.subcore_barrier()

    base = (c * NS + s) * PER_W

    @pl.loop(0, NCHUNK)
    def _(j):
        off = base + j * C
        pltpu.sync_copy(dst_hbm.at[pl.ds(off, C)], idx_v)
        pltpu.sync_copy(msg_hbm.at[pl.ds(off, C)], rows_v)
        pltpu.sync_copy(rows_v, acc_sh.at[idx_v], add=True)

    plsc.subcore_barrier()
    pltpu.sync_copy(
        acc_sh.at[pl.ds(s * ROWS_PER_TILE, ROWS_PER_TILE)],
        out_hbm.at[c].at[pl.ds(s * ROWS_PER_TILE, ROWS_PER_TILE)],
    )


def _sc_scatter(msg, dst):
    mesh = plsc.VectorSubcoreMesh(core_axis_name="c", subcore_axis_name="s")
    z = jnp.zeros((ROWS_PER_TILE, MSG), jnp.float32)
    f = pl.kernel(
        _scatter_body,
        out_type=jax.ShapeDtypeStruct((NC, NAGG, MSG), jnp.float32),
        mesh=mesh,
        scratch_types=[
            pltpu.VMEM((C,), jnp.int32),
            pltpu.VMEM((C, MSG), jnp.float32),
            pltpu.VMEM_SHARED((NAGG, MSG), jnp.float32),
        ],
    )
    return f(msg, dst, z)


def _silu(v):
    return v * jax.nn.sigmoid(v)


def _bdot(a, b):
    return jnp.dot(a.astype(jnp.bfloat16), b.astype(jnp.bfloat16),
                   preferred_element_type=jnp.float32)


def _emlp_body(xi_ref, xj_ref, ea_ref, w1a, w1b, w1c, b1, w2, b2, w3, b3,
               out_ref):
    xi_lo, xi_hi = _unpack(xi_ref[...])
    xj_lo, xj_hi = _unpack(xj_ref[...])
    h = (_bdot(xi_lo, w1a[:NPK]) + _bdot(xi_hi, w1a[NPK:])
         + _bdot(xj_lo, w1b[:NPK]) + _bdot(xj_hi, w1b[NPK:])
         + _bdot(ea_ref[...], w1c[...])
         + b1[...])
    h = _silu(h)
    h = _silu(_bdot(h, w2[...]) + b2[...])
    out_ref[...] = _bdot(h, w3[...]) + b3[...]


def _tc_edge_mlp(xi, xj, ea, mW1, mb1, mW2, mb2, mW3, mb3):
    BE = 1280
    grid = (E // BE,)
    w1a = mW1[:NF]
    w1b = mW1[NF:2 * NF]
    w1c = mW1[2 * NF:]
    full = lambda shape: pl.BlockSpec(shape, lambda i: (0,) * len(shape))
    return pl.pallas_call(
        _emlp_body,
        grid=grid,
        in_specs=[
            pl.BlockSpec((BE, NPK), lambda i: (i, 0)),
            pl.BlockSpec((BE, NPK), lambda i: (i, 0)),
            pl.BlockSpec((BE, NEF), lambda i: (i, 0)),
            full((NF, HID)),
            full((NF, HID)),
            full((NEF, HID)),
            full((1, HID)),
            full((HID, HID)),
            full((1, HID)),
            full((HID, MSG)),
            full((1, MSG)),
        ],
        out_specs=pl.BlockSpec((BE, MSG), lambda i: (i, 0)),
        out_shape=jax.ShapeDtypeStruct((E, MSG), jnp.float32),
    )(xi, xj, ea, w1a, w1b, w1c, mb1.reshape(1, HID), mW2,
      mb2.reshape(1, HID), mW3, mb3.reshape(1, MSG))


def _nmlp_body(aggr2_ref, x_ref, batch_ref, w1a, w1b, b1, w2, b2, w3, b3,
               lw, lb, out_ref, pool_acc, cnt_acc):
    i = pl.program_id(0)
    nb = pl.num_programs(0)

    @pl.when(i == 0)
    def _():
        pool_acc[...] = jnp.zeros_like(pool_acc)
        cnt_acc[...] = jnp.zeros_like(cnt_acc)

    aggr = aggr2_ref[0] + aggr2_ref[1]
    h = (_bdot(x_ref[...], w1a[...])
         + _bdot(aggr, w1b[...])
         + b1[...])
    h = _silu(h)
    h = _silu(_bdot(h, w2[...]) + b2[...])
    h = _bdot(h, w3[...]) + b3[...]

    ids = batch_ref[...].reshape(1, -1)
    iota = lax.broadcasted_iota(jnp.int32, (NG, ids.shape[1]), 0)
    onehot = (iota == ids).astype(jnp.float32)
    pool_acc[...] += jnp.dot(onehot, h, preferred_element_type=jnp.float32)
    cnt = jnp.sum(onehot, axis=1, keepdims=True)
    cnt_acc[...] += jnp.broadcast_to(cnt, cnt_acc.shape)

    @pl.when(i == nb - 1)
    def _():
        pooled = pool_acc[...] / jnp.maximum(cnt_acc[...], 1.0)
        out_ref[...] = (
            jnp.dot(pooled, lw[...], preferred_element_type=jnp.float32)
            + lb[...])


def _tc_node_mlp(aggr2, x, batch, nW1, nb1, nW2, nb2, nW3, nb3, lW, lb):
    BN = 400
    nblocks = N // BN
    batch3 = batch.reshape(nblocks, 1, BN)
    w1a = nW1[:NF]
    w1b = nW1[NF:]
    full = lambda shape: pl.BlockSpec(shape, lambda i: (0,) * len(shape))
    return pl.pallas_call(
        _nmlp_body,
        grid=(nblocks,),
        in_specs=[
            pl.BlockSpec((NC, BN, MSG), lambda i: (0, i, 0)),
            pl.BlockSpec((BN, NF), lambda i: (i, 0)),
            pl.BlockSpec((1, 1, BN), lambda i: (i, 0, 0)),
            full((NF, HID)),
            full((MSG, HID)),
            full((1, HID)),
            full((HID, HID)),
            full((1, HID)),
            full((HID, NH)),
            full((1, NH)),
            full((NH, NP)),
            full((1, NP)),
        ],
        out_specs=pl.BlockSpec((NG, NP), lambda i: (0, 0)),
        out_shape=jax.ShapeDtypeStruct((NG, NP), jnp.float32),
        scratch_shapes=[
            pltpu.VMEM((NG, NH), jnp.float32),
            pltpu.VMEM((NG, NH), jnp.float32),
        ],
    )(aggr2, x, batch3, w1a, w1b, nb1.reshape(1, HID), nW2,
      nb2.reshape(1, HID), nW3, nb3.reshape(1, NH), lW, lb.reshape(1, NP))


def kernel(x, edge_index, edge_attr, batch,
           mW1, mb1, mW2, mb2, mW3, mb3,
           nW1, nb1, nW2, nb2, nW3, nb3,
           lW, lb):
    src = edge_index[0]
    dst = edge_index[1]
    xj, xi = _sc_gather(_pack_x(x), src, dst)
    msg = _tc_edge_mlp(xi, xj, edge_attr, mW1, mb1, mW2, mb2, mW3, mb3)
    aggr2 = _sc_scatter(msg, dst)
    return _tc_node_mlp(aggr2, x, batch, nW1, nb1, nW2, nb2, nW3, nb3, lW, lb)


# speedup vs baseline: 1.3654x; 1.3654x over previous
import functools

import jax
import jax.numpy as jnp
from jax import lax
from jax.experimental import pallas as pl
from jax.experimental.pallas import tpu as pltpu
from jax.experimental.pallas import tpu_sc as plsc

N = 10000
E = 320000
NF = 128
NEF = 16
MSG = 128
HID = 300
NH = 128
NP = 2
NG = 64

NC = 2
NS = 16
NW = NC * NS

NSLICE = 2
ES = E // NSLICE
PER_W = ES // NW
C = 40
NCHUNK = PER_W // C
NAGG = 10240
ROWS_PER_TILE = NAGG // NS


def _gather_body(soff, x_hbm, src_hbm, dst_hbm, xj_hbm, xi_hbm,
                 idx_a, rows_a, idx_b, rows_b, sem_a, sem_b):
    c = lax.axis_index("c")
    s = lax.axis_index("s")
    wbase = (c * NS + s) * PER_W

    @pl.loop(0, NCHUNK)
    def _(j):
        off = wbase + j * C
        pltpu.sync_copy(src_hbm.at[pl.ds(soff + off, C)], idx_a)
        cp_a = pltpu.async_copy(x_hbm.at[idx_a], rows_a, sem_a)
        pltpu.sync_copy(dst_hbm.at[pl.ds(soff + off, C)], idx_b)
        cp_b = pltpu.async_copy(x_hbm.at[idx_b], rows_b, sem_b)
        cp_a.wait()
        pltpu.sync_copy(rows_a, xj_hbm.at[pl.ds(off, C)])
        cp_b.wait()
        pltpu.sync_copy(rows_b, xi_hbm.at[pl.ds(off, C)])


def _sc_gather(x, src, dst, sl):
    mesh = plsc.VectorSubcoreMesh(core_axis_name="c", subcore_axis_name="s")
    f = pl.kernel(
        functools.partial(_gather_body, sl * ES),
        out_type=(
            jax.ShapeDtypeStruct((ES, NF), jnp.float32),
            jax.ShapeDtypeStruct((ES, NF), jnp.float32),
        ),
        mesh=mesh,
        scratch_types=[
            pltpu.VMEM((C,), jnp.int32),
            pltpu.VMEM((C, NF), jnp.float32),
            pltpu.VMEM((C,), jnp.int32),
            pltpu.VMEM((C, NF), jnp.float32),
            pltpu.SemaphoreType.DMA,
            pltpu.SemaphoreType.DMA,
        ],
        name=f"sc_gather_{sl}",
    )
    return f(x, src, dst)


def _scatter_body(soff, msg_hbm, dst_hbm, z_hbm, out_hbm, idx_v, rows_v,
                  acc_sh):
    c = lax.axis_index("c")
    s = lax.axis_index("s")
    pltpu.sync_copy(z_hbm, acc_sh.at[pl.ds(s * ROWS_PER_TILE, ROWS_PER_TILE)])
    plsc.subcore_barrier()

    wbase = (c * NS + s) * PER_W

    @pl.loop(0, NCHUNK)
    def _(j):
        off = wbase + j * C
        pltpu.sync_copy(dst_hbm.at[pl.ds(soff + off, C)], idx_v)
        pltpu.sync_copy(msg_hbm.at[pl.ds(off, C)], rows_v)
        pltpu.sync_copy(rows_v, acc_sh.at[idx_v], add=True)

    plsc.subcore_barrier()
    pltpu.sync_copy(
        acc_sh.at[pl.ds(s * ROWS_PER_TILE, ROWS_PER_TILE)],
        out_hbm.at[c].at[pl.ds(s * ROWS_PER_TILE, ROWS_PER_TILE)],
    )


def _sc_scatter(msg, dst, sl):
    mesh = plsc.VectorSubcoreMesh(core_axis_name="c", subcore_axis_name="s")
    z = jnp.zeros((ROWS_PER_TILE, MSG), jnp.float32)
    f = pl.kernel(
        functools.partial(_scatter_body, sl * ES),
        out_type=jax.ShapeDtypeStruct((NC, NAGG, MSG), jnp.float32),
        mesh=mesh,
        scratch_types=[
            pltpu.VMEM((C,), jnp.int32),
            pltpu.VMEM((C, MSG), jnp.float32),
            pltpu.VMEM_SHARED((NAGG, MSG), jnp.float32),
        ],
        name=f"sc_scatter_{sl}",
    )
    return f(msg, dst, z)


def _silu(v):
    return v * jax.nn.sigmoid(v)


def _bdot(a, b):
    return jnp.dot(a.astype(jnp.bfloat16), b.astype(jnp.bfloat16),
                   preferred_element_type=jnp.float32)


def _emlp_body(xi_ref, xj_ref, ea_ref, w1a, w1b, w1c, b1, w2, b2, w3, b3,
               out_ref):
    h = (_bdot(xi_ref[...], w1a[...])
         + _bdot(xj_ref[...], w1b[...])
         + _bdot(ea_ref[...], w1c[...])
         + b1[...])
    h = _silu(h)
    h = _silu(_bdot(h, w2[...]) + b2[...])
    out_ref[...] = _bdot(h, w3[...]) + b3[...]


def _tc_edge_mlp(xi, xj, ea, sl, mW1, mb1, mW2, mb2, mW3, mb3):
    BE = 1280
    grid = (ES // BE,)
    ea_off = sl * (ES // BE)
    w1a = mW1[:NF]
    w1b = mW1[NF:2 * NF]
    w1c = mW1[2 * NF:]
    full = lambda shape: pl.BlockSpec(shape, lambda i: (0,) * len(shape))
    return pl.pallas_call(
        _emlp_body,
        grid=grid,
        in_specs=[
            pl.BlockSpec((BE, NF), lambda i: (i, 0)),
            pl.BlockSpec((BE, NF), lambda i: (i, 0)),
            pl.BlockSpec((BE, NEF), lambda i: (i + ea_off, 0)),
            full((NF, HID)),
            full((NF, HID)),
            full((NEF, HID)),
            full((1, HID)),
            full((HID, HID)),
            full((1, HID)),
            full((HID, MSG)),
            full((1, MSG)),
        ],
        out_specs=pl.BlockSpec((BE, MSG), lambda i: (i, 0)),
        out_shape=jax.ShapeDtypeStruct((ES, MSG), jnp.float32),
        name=f"tc_edge_mlp_{sl}",
    )(xi, xj, ea, w1a, w1b, w1c, mb1.reshape(1, HID), mW2,
      mb2.reshape(1, HID), mW3, mb3.reshape(1, MSG))


def _nmlp_body(a0_ref, a1_ref, x_ref, batch_ref, w1a, w1b, b1, w2, b2, w3,
               b3, lw, lb, out_ref, pool_acc, cnt_acc):
    i = pl.program_id(0)
    nb = pl.num_programs(0)

    @pl.when(i == 0)
    def _():
        pool_acc[...] = jnp.zeros_like(pool_acc)
        cnt_acc[...] = jnp.zeros_like(cnt_acc)

    aggr = (a0_ref[0] + a0_ref[1]) + (a1_ref[0] + a1_ref[1])
    h = (_bdot(x_ref[...], w1a[...])
         + _bdot(aggr, w1b[...])
         + b1[...])
    h = _silu(h)
    h = _silu(_bdot(h, w2[...]) + b2[...])
    h = _bdot(h, w3[...]) + b3[...]

    ids = batch_ref[...].reshape(1, -1)
    iota = lax.broadcasted_iota(jnp.int32, (NG, ids.shape[1]), 0)
    onehot = (iota == ids).astype(jnp.float32)
    pool_acc[...] += jnp.dot(onehot, h, preferred_element_type=jnp.float32)
    cnt = jnp.sum(onehot, axis=1, keepdims=True)
    cnt_acc[...] += jnp.broadcast_to(cnt, cnt_acc.shape)

    @pl.when(i == nb - 1)
    def _():
        pooled = pool_acc[...] / jnp.maximum(cnt_acc[...], 1.0)
        out_ref[...] = (
            jnp.dot(pooled, lw[...], preferred_element_type=jnp.float32)
            + lb[...])


def _tc_node_mlp(a0, a1, x, batch, nW1, nb1, nW2, nb2, nW3, nb3, lW, lb):
    BN = 400
    nblocks = N // BN
    batch3 = batch.reshape(nblocks, 1, BN)
    w1a = nW1[:NF]
    w1b = nW1[NF:]
    full = lambda shape: pl.BlockSpec(shape, lambda i: (0,) * len(shape))
    return pl.pallas_call(
        _nmlp_body,
        grid=(nblocks,),
        in_specs=[
            pl.BlockSpec((NC, BN, MSG), lambda i: (0, i, 0)),
            pl.BlockSpec((NC, BN, MSG), lambda i: (0, i, 0)),
            pl.BlockSpec((BN, NF), lambda i: (i, 0)),
            pl.BlockSpec((1, 1, BN), lambda i: (i, 0, 0)),
            full((NF, HID)),
            full((MSG, HID)),
            full((1, HID)),
            full((HID, HID)),
            full((1, HID)),
            full((HID, NH)),
            full((1, NH)),
            full((NH, NP)),
            full((1, NP)),
        ],
        out_specs=pl.BlockSpec((NG, NP), lambda i: (0, 0)),
        out_shape=jax.ShapeDtypeStruct((NG, NP), jnp.float32),
        scratch_shapes=[
            pltpu.VMEM((NG, NH), jnp.float32),
            pltpu.VMEM((NG, NH), jnp.float32),
        ],
        name="tc_node_mlp",
    )(a0, a1, x, batch3, w1a, w1b, nb1.reshape(1, HID), nW2,
      nb2.reshape(1, HID), nW3, nb3.reshape(1, NH), lW, lb.reshape(1, NP))


def kernel(x, edge_index, edge_attr, batch,
           mW1, mb1, mW2, mb2, mW3, mb3,
           nW1, nb1, nW2, nb2, nW3, nb3,
           lW, lb):
    src = edge_index[0]
    dst = edge_index[1]
    xj0, xi0 = _sc_gather(x, src, dst, 0)
    xj1, xi1 = _sc_gather(x, src, dst, 1)
    msg0 = _tc_edge_mlp(xi0, xj0, edge_attr, 0, mW1, mb1, mW2, mb2, mW3, mb3)
    msg1 = _tc_edge_mlp(xi1, xj1, edge_attr, 1, mW1, mb1, mW2, mb2, mW3, mb3)
    a0 = _sc_scatter(msg0, dst, 0)
    a1 = _sc_scatter(msg1, dst, 1)
    return _tc_node_mlp(a0, a1, x, batch, nW1, nb1, nW2, nb2, nW3, nb3,
                        lW, lb)


# speedup vs baseline: 1.7414x; 1.2754x over previous
import functools

import jax
import jax.numpy as jnp
from jax import lax
from jax.experimental import pallas as pl
from jax.experimental.pallas import tpu as pltpu
from jax.experimental.pallas import tpu_sc as plsc

N = 10000
E = 320000
NF = 128
NEF = 16
MSG = 128
HID = 300
NH = 128
NP = 2
NG = 64

NC = 2
NS = 16
NW = NC * NS

C = 80
PW = (E // NW) // 2
PW0 = 4800
PW1 = 2 * PW - PW0
ES0 = PW0 * NW
ES1 = PW1 * NW
NAGG = 10240
ROWS_PER_TILE = NAGG // NS


def _maybe_when(cond, fn):
    if isinstance(cond, bool):
        if cond:
            fn()
    else:
        pl.when(cond)(fn)


def _gather_body(goff, per_w, x_hbm, src_hbm, dst_hbm, xj_hbm, xi_hbm,
                 idx_s, idx_d, rows_s, rows_d, sem_is, sem_id, sem_g, sem_w):
    c = lax.axis_index("c")
    s = lax.axis_index("s")
    wbase = (c * NS + s) * per_w
    nchunks = per_w // C

    def stage(j, b):
        o = wbase + j * C
        g = goff + o

        def drain_prev_wb():
            pltpu.make_async_copy(rows_s[b], xj_hbm.at[pl.ds(0, C)],
                                  sem_w[b]).wait()
            pltpu.make_async_copy(rows_d[b], xi_hbm.at[pl.ds(0, C)],
                                  sem_w[b]).wait()

        _maybe_when(j >= 2, drain_prev_wb)

        ci = pltpu.async_copy(src_hbm.at[pl.ds(g, C)], idx_s[b], sem_is[b])
        cd = pltpu.async_copy(dst_hbm.at[pl.ds(g, C)], idx_d[b], sem_id[b])
        ci.wait()
        pltpu.async_copy(x_hbm.at[idx_s[b]], rows_s[b], sem_g[b])
        cd.wait()
        pltpu.async_copy(x_hbm.at[idx_d[b]], rows_d[b], sem_g[b])

        def wb_prev():
            ob = wbase + (j - 1) * C
            p = 1 - b
            pltpu.make_async_copy(x_hbm.at[idx_s[p]], rows_s[p],
                                  sem_g[p]).wait()
            pltpu.make_async_copy(x_hbm.at[idx_d[p]], rows_d[p],
                                  sem_g[p]).wait()
            pltpu.async_copy(rows_s[p], xj_hbm.at[pl.ds(ob, C)], sem_w[p])
            pltpu.async_copy(rows_d[p], xi_hbm.at[pl.ds(ob, C)], sem_w[p])

        _maybe_when(j >= 1, wb_prev)

    @pl.loop(0, nchunks // 2)
    def _(i):
        stage(2 * i, 0)
        stage(2 * i + 1, 1)

    if nchunks % 2:
        stage(nchunks - 1, 0)
    bl = (nchunks - 1) % 2
    ol = wbase + (nchunks - 1) * C
    pltpu.make_async_copy(x_hbm.at[idx_s[bl]], rows_s[bl], sem_g[bl]).wait()
    pltpu.make_async_copy(x_hbm.at[idx_d[bl]], rows_d[bl], sem_g[bl]).wait()
    pltpu.sync_copy(rows_s[bl], xj_hbm.at[pl.ds(ol, C)])
    pltpu.sync_copy(rows_d[bl], xi_hbm.at[pl.ds(ol, C)])
    pltpu.make_async_copy(rows_s[1 - bl], xj_hbm.at[pl.ds(0, C)],
                          sem_w[1 - bl]).wait()
    pltpu.make_async_copy(rows_d[1 - bl], xi_hbm.at[pl.ds(0, C)],
                          sem_w[1 - bl]).wait()


def _sc_gather(x, src, dst, sl):
    goff, per_w, es = (0, PW0, ES0) if sl == 0 else (ES0, PW1, ES1)
    mesh = plsc.VectorSubcoreMesh(core_axis_name="c", subcore_axis_name="s")
    f = pl.kernel(
        functools.partial(_gather_body, goff, per_w),
        out_type=(
            jax.ShapeDtypeStruct((es, NF), jnp.float32),
            jax.ShapeDtypeStruct((es, NF), jnp.float32),
        ),
        mesh=mesh,
        scratch_types=[
            [pltpu.VMEM((C,), jnp.int32)] * 2,
            [pltpu.VMEM((C,), jnp.int32)] * 2,
            [pltpu.VMEM((C, NF), jnp.float32)] * 2,
            [pltpu.VMEM((C, NF), jnp.float32)] * 2,
            [pltpu.SemaphoreType.DMA] * 2,
            [pltpu.SemaphoreType.DMA] * 2,
            [pltpu.SemaphoreType.DMA] * 2,
            [pltpu.SemaphoreType.DMA] * 2,
        ],
        name=f"sc_gather_{sl}",
    )
    return f(x, src, dst)


def _scatter_body(goff, per_w, msg_hbm, dst_hbm, z_hbm, out_hbm,
                  idx_v, rows_v, acc_sh, sem_i, sem_m, sem_a):
    c = lax.axis_index("c")
    s = lax.axis_index("s")
    pltpu.sync_copy(z_hbm, acc_sh.at[pl.ds(s * ROWS_PER_TILE, ROWS_PER_TILE)])
    plsc.subcore_barrier()

    wbase = (c * NS + s) * per_w
    nchunks = per_w // C

    def stage(j, b):
        o = wbase + j * C
        g = goff + o

        def drain_prev_add():
            pltpu.make_async_copy(rows_v[b], acc_sh.at[idx_v[b]],
                                  sem_a[b]).wait()

        _maybe_when(j >= 2, drain_prev_add)

        ci = pltpu.async_copy(dst_hbm.at[pl.ds(g, C)], idx_v[b], sem_i[b])
        cm = pltpu.async_copy(msg_hbm.at[pl.ds(o, C)], rows_v[b], sem_m[b])
        ci.wait()
        cm.wait()
        pltpu.async_copy(rows_v[b], acc_sh.at[idx_v[b]], sem_a[b], add=True)

    @pl.loop(0, nchunks // 2)
    def _(i):
        stage(2 * i, 0)
        stage(2 * i + 1, 1)

    if nchunks % 2:
        stage(nchunks - 1, 0)
    bl = (nchunks - 1) % 2
    pltpu.make_async_copy(rows_v[bl], acc_sh.at[idx_v[bl]], sem_a[bl]).wait()
    if nchunks >= 2:
        pltpu.make_async_copy(rows_v[1 - bl], acc_sh.at[idx_v[1 - bl]],
                              sem_a[1 - bl]).wait()

    plsc.subcore_barrier()
    pltpu.sync_copy(
        acc_sh.at[pl.ds(s * ROWS_PER_TILE, ROWS_PER_TILE)],
        out_hbm.at[c].at[pl.ds(s * ROWS_PER_TILE, ROWS_PER_TILE)],
    )


def _sc_scatter(msg, dst, sl):
    goff, per_w = (0, PW0) if sl == 0 else (ES0, PW1)
    mesh = plsc.VectorSubcoreMesh(core_axis_name="c", subcore_axis_name="s")
    z = jnp.zeros((ROWS_PER_TILE, MSG), jnp.float32)
    f = pl.kernel(
        functools.partial(_scatter_body, goff, per_w),
        out_type=jax.ShapeDtypeStruct((NC, NAGG, MSG), jnp.float32),
        mesh=mesh,
        scratch_types=[
            [pltpu.VMEM((C,), jnp.int32)] * 2,
            [pltpu.VMEM((C, MSG), jnp.float32)] * 2,
            pltpu.VMEM_SHARED((NAGG, MSG), jnp.float32),
            [pltpu.SemaphoreType.DMA] * 2,
            [pltpu.SemaphoreType.DMA] * 2,
            [pltpu.SemaphoreType.DMA] * 2,
        ],
        name=f"sc_scatter_{sl}",
    )
    return f(msg, dst, z)


def _silu(v):
    return v * jax.nn.sigmoid(v)


def _bdot(a, b):
    return jnp.dot(a.astype(jnp.bfloat16), b.astype(jnp.bfloat16),
                   preferred_element_type=jnp.float32)


def _emlp_body(xi_ref, xj_ref, ea_ref, w1a, w1b, w1c, b1, w2, b2, w3, b3,
               out_ref):
    h = (_bdot(xi_ref[...], w1a[...])
         + _bdot(xj_ref[...], w1b[...])
         + _bdot(ea_ref[...], w1c[...])
         + b1[...])
    h = _silu(h)
    h = _silu(_bdot(h, w2[...]) + b2[...])
    out_ref[...] = _bdot(h, w3[...]) + b3[...]


def _tc_edge_mlp(xi, xj, ea, sl, mW1, mb1, mW2, mb2, mW3, mb3):
    BE = 1280
    es = xi.shape[0]
    grid = (es // BE,)
    ea_off = 0 if sl == 0 else ES0 // BE
    w1a = mW1[:NF]
    w1b = mW1[NF:2 * NF]
    w1c = mW1[2 * NF:]
    full = lambda shape: pl.BlockSpec(shape, lambda i: (0,) * len(shape))
    return pl.pallas_call(
        _emlp_body,
        grid=grid,
        in_specs=[
            pl.BlockSpec((BE, NF), lambda i: (i, 0)),
            pl.BlockSpec((BE, NF), lambda i: (i, 0)),
            pl.BlockSpec((BE, NEF), lambda i: (i + ea_off, 0)),
            full((NF, HID)),
            full((NF, HID)),
            full((NEF, HID)),
            full((1, HID)),
            full((HID, HID)),
            full((1, HID)),
            full((HID, MSG)),
            full((1, MSG)),
        ],
        out_specs=pl.BlockSpec((BE, MSG), lambda i: (i, 0)),
        out_shape=jax.ShapeDtypeStruct((es, MSG), jnp.float32),
        name=f"tc_edge_mlp_{sl}",
    )(xi, xj, ea, w1a, w1b, w1c, mb1.reshape(1, HID), mW2,
      mb2.reshape(1, HID), mW3, mb3.reshape(1, MSG))


def _nmlp_body(a0_ref, a1_ref, x_ref, batch_ref, w1a, w1b, b1, w2, b2, w3,
               b3, lw, lb, out_ref, pool_acc, cnt_acc):
    i = pl.program_id(0)
    nb = pl.num_programs(0)

    @pl.when(i == 0)
    def _():
        pool_acc[...] = jnp.zeros_like(pool_acc)
        cnt_acc[...] = jnp.zeros_like(cnt_acc)

    aggr = (a0_ref[0] + a0_ref[1]) + (a1_ref[0] + a1_ref[1])
    h = (_bdot(x_ref[...], w1a[...])
         + _bdot(aggr, w1b[...])
         + b1[...])
    h = _silu(h)
    h = _silu(_bdot(h, w2[...]) + b2[...])
    h = _bdot(h, w3[...]) + b3[...]

    ids = batch_ref[...].reshape(1, -1)
    iota = lax.broadcasted_iota(jnp.int32, (NG, ids.shape[1]), 0)
    onehot = (iota == ids).astype(jnp.float32)
    pool_acc[...] += jnp.dot(onehot, h, preferred_element_type=jnp.float32)
    cnt = jnp.sum(onehot, axis=1, keepdims=True)
    cnt_acc[...] += jnp.broadcast_to(cnt, cnt_acc.shape)

    @pl.when(i == nb - 1)
    def _():
        pooled = pool_acc[...] / jnp.maximum(cnt_acc[...], 1.0)
        out_ref[...] = (
            jnp.dot(pooled, lw[...], preferred_element_type=jnp.float32)
            + lb[...])


def _tc_node_mlp(a0, a1, x, batch, nW1, nb1, nW2, nb2, nW3, nb3, lW, lb):
    BN = 400
    nblocks = N // BN
    batch3 = batch.reshape(nblocks, 1, BN)
    w1a = nW1[:NF]
    w1b = nW1[NF:]
    full = lambda shape: pl.BlockSpec(shape, lambda i: (0,) * len(shape))
    return pl.pallas_call(
        _nmlp_body,
        grid=(nblocks,),
        in_specs=[
            pl.BlockSpec((NC, BN, MSG), lambda i: (0, i, 0)),
            pl.BlockSpec((NC, BN, MSG), lambda i: (0, i, 0)),
            pl.BlockSpec((BN, NF), lambda i: (i, 0)),
            pl.BlockSpec((1, 1, BN), lambda i: (i, 0, 0)),
            full((NF, HID)),
            full((MSG, HID)),
            full((1, HID)),
            full((HID, HID)),
            full((1, HID)),
            full((HID, NH)),
            full((1, NH)),
            full((NH, NP)),
            full((1, NP)),
        ],
        out_specs=pl.BlockSpec((NG, NP), lambda i: (0, 0)),
        out_shape=jax.ShapeDtypeStruct((NG, NP), jnp.float32),
        scratch_shapes=[
            pltpu.VMEM((NG, NH), jnp.float32),
            pltpu.VMEM((NG, NH), jnp.float32),
        ],
        name="tc_node_mlp",
    )(a0, a1, x, batch3, w1a, w1b, nb1.reshape(1, HID), nW2,
      nb2.reshape(1, HID), nW3, nb3.reshape(1, NH), lW, lb.reshape(1, NP))


def kernel(x, edge_index, edge_attr, batch,
           mW1, mb1, mW2, mb2, mW3, mb3,
           nW1, nb1, nW2, nb2, nW3, nb3,
           lW, lb):
    src = edge_index[0]
    dst = edge_index[1]
    xj0, xi0 = _sc_gather(x, src, dst, 0)
    xj1, xi1 = _sc_gather(x, src, dst, 1)
    msg0 = _tc_edge_mlp(xi0, xj0, edge_attr, 0, mW1, mb1, mW2, mb2, mW3, mb3)
    msg1 = _tc_edge_mlp(xi1, xj1, edge_attr, 1, mW1, mb1, mW2, mb2, mW3, mb3)
    a0 = _sc_scatter(msg0, dst, 0)
    a1 = _sc_scatter(msg1, dst, 1)
    return _tc_node_mlp(a0, a1, x, batch, nW1, nb1, nW2, nb2, nW3, nb3,
                        lW, lb)


# speedup vs baseline: 1.8931x; 1.0871x over previous
import functools

import jax
import jax.numpy as jnp
from jax import lax
from jax.experimental import pallas as pl
from jax.experimental.pallas import tpu as pltpu
from jax.experimental.pallas import tpu_sc as plsc

N = 10000
E = 320000
NF = 128
NEF = 16
MSG = 128
HID = 300
NH = 128
NP = 2
NG = 64

NC = 2
NS = 16
NW = NC * NS

C = 80
PW = (E // NW) // 2
PW0 = 4800
PW1 = 2 * PW - PW0
ES0 = PW0 * NW
ES1 = PW1 * NW
NAGG = 10240
ROWS_PER_TILE = NAGG // NS


def _maybe_when(cond, fn):
    if isinstance(cond, bool):
        if cond:
            fn()
    else:
        pl.when(cond)(fn)


def _gather_body(goff, per_w, x_hbm, src_hbm, dst_hbm, xj_hbm, xi_hbm,
                 idx_s, idx_d, rows_s, rows_d, sem_is, sem_id, sem_g, sem_w):
    c = lax.axis_index("c")
    s = lax.axis_index("s")
    wbase = (c * NS + s) * per_w
    nchunks = per_w // C

    def stage(j, b):
        o = wbase + j * C
        g = goff + o

        def drain_prev_wb():
            pltpu.make_async_copy(rows_s[b], xj_hbm.at[pl.ds(0, C)],
                                  sem_w[b]).wait()
            pltpu.make_async_copy(rows_d[b], xi_hbm.at[pl.ds(0, C)],
                                  sem_w[b]).wait()

        _maybe_when(j >= 2, drain_prev_wb)

        ci = pltpu.async_copy(src_hbm.at[pl.ds(g, C)], idx_s[b], sem_is[b])
        cd = pltpu.async_copy(dst_hbm.at[pl.ds(g, C)], idx_d[b], sem_id[b])
        ci.wait()
        pltpu.async_copy(x_hbm.at[idx_s[b]], rows_s[b], sem_g[b])
        cd.wait()
        pltpu.async_copy(x_hbm.at[idx_d[b]], rows_d[b], sem_g[b])

        def wb_prev():
            ob = wbase + (j - 1) * C
            p = 1 - b
            pltpu.make_async_copy(x_hbm.at[idx_s[p]], rows_s[p],
                                  sem_g[p]).wait()
            pltpu.make_async_copy(x_hbm.at[idx_d[p]], rows_d[p],
                                  sem_g[p]).wait()
            pltpu.async_copy(rows_s[p], xj_hbm.at[pl.ds(ob, C)], sem_w[p])
            pltpu.async_copy(rows_d[p], xi_hbm.at[pl.ds(ob, C)], sem_w[p])

        _maybe_when(j >= 1, wb_prev)

    @pl.loop(0, nchunks // 2)
    def _(i):
        stage(2 * i, 0)
        stage(2 * i + 1, 1)

    if nchunks % 2:
        stage(nchunks - 1, 0)
    bl = (nchunks - 1) % 2
    ol = wbase + (nchunks - 1) * C
    pltpu.make_async_copy(x_hbm.at[idx_s[bl]], rows_s[bl], sem_g[bl]).wait()
    pltpu.make_async_copy(x_hbm.at[idx_d[bl]], rows_d[bl], sem_g[bl]).wait()
    pltpu.sync_copy(rows_s[bl], xj_hbm.at[pl.ds(ol, C)])
    pltpu.sync_copy(rows_d[bl], xi_hbm.at[pl.ds(ol, C)])
    pltpu.make_async_copy(rows_s[1 - bl], xj_hbm.at[pl.ds(0, C)],
                          sem_w[1 - bl]).wait()
    pltpu.make_async_copy(rows_d[1 - bl], xi_hbm.at[pl.ds(0, C)],
                          sem_w[1 - bl]).wait()


def _sc_gather(x, src, dst, sl):
    goff, per_w, es = (0, PW0, ES0) if sl == 0 else (ES0, PW1, ES1)
    mesh = plsc.VectorSubcoreMesh(core_axis_name="c", subcore_axis_name="s")
    f = pl.kernel(
        functools.partial(_gather_body, goff, per_w),
        out_type=(
            jax.ShapeDtypeStruct((es, NF), jnp.float32),
            jax.ShapeDtypeStruct((es, NF), jnp.float32),
        ),
        mesh=mesh,
        scratch_types=[
            [pltpu.VMEM((C,), jnp.int32)] * 2,
            [pltpu.VMEM((C,), jnp.int32)] * 2,
            [pltpu.VMEM((C, NF), jnp.float32)] * 2,
            [pltpu.VMEM((C, NF), jnp.float32)] * 2,
            [pltpu.SemaphoreType.DMA] * 2,
            [pltpu.SemaphoreType.DMA] * 2,
            [pltpu.SemaphoreType.DMA] * 2,
            [pltpu.SemaphoreType.DMA] * 2,
        ],
        name=f"sc_gather_{sl}",
    )
    return f(x, src, dst)


def _scatter_body(goff, per_w, msg_hbm, dst_hbm, z_hbm, out_hbm,
                  idx_v, rows_v, acc_sh, sem_i, sem_m, sem_a):
    c = lax.axis_index("c")
    s = lax.axis_index("s")
    pltpu.sync_copy(z_hbm, acc_sh.at[pl.ds(s * ROWS_PER_TILE, ROWS_PER_TILE)])
    plsc.subcore_barrier()

    wbase = (c * NS + s) * per_w
    nchunks = per_w // C

    def stage(j, b):
        o = wbase + j * C
        g = goff + o

        def drain_prev_add():
            pltpu.make_async_copy(rows_v[b], acc_sh.at[idx_v[b]],
                                  sem_a[b]).wait()

        _maybe_when(j >= 2, drain_prev_add)

        ci = pltpu.async_copy(dst_hbm.at[pl.ds(g, C)], idx_v[b], sem_i[b])
        cm = pltpu.async_copy(msg_hbm.at[pl.ds(o, C)], rows_v[b], sem_m[b])
        ci.wait()
        cm.wait()
        pltpu.async_copy(rows_v[b], acc_sh.at[idx_v[b]], sem_a[b], add=True)

    @pl.loop(0, nchunks // 2)
    def _(i):
        stage(2 * i, 0)
        stage(2 * i + 1, 1)

    if nchunks % 2:
        stage(nchunks - 1, 0)
    bl = (nchunks - 1) % 2
    pltpu.make_async_copy(rows_v[bl], acc_sh.at[idx_v[bl]], sem_a[bl]).wait()
    if nchunks >= 2:
        pltpu.make_async_copy(rows_v[1 - bl], acc_sh.at[idx_v[1 - bl]],
                              sem_a[1 - bl]).wait()

    plsc.subcore_barrier()
    pltpu.sync_copy(
        acc_sh.at[pl.ds(s * ROWS_PER_TILE, ROWS_PER_TILE)],
        out_hbm.at[c].at[pl.ds(s * ROWS_PER_TILE, ROWS_PER_TILE)],
    )


def _sc_scatter(msg, dst, sl):
    goff, per_w = (0, PW0) if sl == 0 else (ES0, PW1)
    mesh = plsc.VectorSubcoreMesh(core_axis_name="c", subcore_axis_name="s")
    z = jnp.zeros((ROWS_PER_TILE, MSG), jnp.float32)
    f = pl.kernel(
        functools.partial(_scatter_body, goff, per_w),
        out_type=jax.ShapeDtypeStruct((NC, NAGG, MSG), jnp.float32),
        mesh=mesh,
        scratch_types=[
            [pltpu.VMEM((C,), jnp.int32)] * 2,
            [pltpu.VMEM((C, MSG), jnp.float32)] * 2,
            pltpu.VMEM_SHARED((NAGG, MSG), jnp.float32),
            [pltpu.SemaphoreType.DMA] * 2,
            [pltpu.SemaphoreType.DMA] * 2,
            [pltpu.SemaphoreType.DMA] * 2,
        ],
        name=f"sc_scatter_{sl}",
    )
    return f(msg, dst, z)


def _silu(v):
    return v * jax.nn.sigmoid(v)


def _bdot(a, b):
    return jnp.dot(a.astype(jnp.bfloat16), b.astype(jnp.bfloat16),
                   preferred_element_type=jnp.float32)


def _emlp_body(xi_ref, xj_ref, eat_ref, w1a, w1b, w1c, b1, w2, b2, w3, b3,
               out_ref):
    eac = lax.dot_general(
        eat_ref[...].astype(jnp.bfloat16), w1c[...].astype(jnp.bfloat16),
        dimension_numbers=(((0,), (0,)), ((), ())),
        preferred_element_type=jnp.float32)
    h = (_bdot(xi_ref[...], w1a[...])
         + _bdot(xj_ref[...], w1b[...])
         + eac
         + b1[...])
    h = _silu(h)
    h = _silu(_bdot(h, w2[...]) + b2[...])
    out_ref[...] = _bdot(h, w3[...]) + b3[...]


def _tc_edge_mlp(xi, xj, ea_t, sl, mW1, mb1, mW2, mb2, mW3, mb3):
    BE = 1280
    es = xi.shape[0]
    grid = (es // BE,)
    ea_off = 0 if sl == 0 else ES0 // BE
    w1a = mW1[:NF]
    w1b = mW1[NF:2 * NF]
    w1c = mW1[2 * NF:]
    full = lambda shape: pl.BlockSpec(shape, lambda i: (0,) * len(shape))
    return pl.pallas_call(
        _emlp_body,
        grid=grid,
        in_specs=[
            pl.BlockSpec((BE, NF), lambda i: (i, 0)),
            pl.BlockSpec((BE, NF), lambda i: (i, 0)),
            pl.BlockSpec((NEF, BE), lambda i: (0, i + ea_off)),
            full((NF, HID)),
            full((NF, HID)),
            full((NEF, HID)),
            full((1, HID)),
            full((HID, HID)),
            full((1, HID)),
            full((HID, MSG)),
            full((1, MSG)),
        ],
        out_specs=pl.BlockSpec((BE, MSG), lambda i: (i, 0)),
        out_shape=jax.ShapeDtypeStruct((es, MSG), jnp.float32),
        name=f"tc_edge_mlp_{sl}",
    )(xi, xj, ea_t, w1a, w1b, w1c, mb1.reshape(1, HID), mW2,
      mb2.reshape(1, HID), mW3, mb3.reshape(1, MSG))


def _nmlp_body(a0_ref, a1_ref, x_ref, batch_ref, w1a, w1b, b1, w2, b2, w3,
               b3, lw, lb, out_ref, pool_acc, cnt_acc):
    i = pl.program_id(0)
    nb = pl.num_programs(0)

    @pl.when(i == 0)
    def _():
        pool_acc[...] = jnp.zeros_like(pool_acc)
        cnt_acc[...] = jnp.zeros_like(cnt_acc)

    aggr = (a0_ref[0] + a0_ref[1]) + (a1_ref[0] + a1_ref[1])
    h = (_bdot(x_ref[...], w1a[...])
         + _bdot(aggr, w1b[...])
         + b1[...])
    h = _silu(h)
    h = _silu(_bdot(h, w2[...]) + b2[...])
    h = _bdot(h, w3[...]) + b3[...]

    ids = batch_ref[...].reshape(1, -1)
    iota = lax.broadcasted_iota(jnp.int32, (NG, ids.shape[1]), 0)
    onehot = (iota == ids).astype(jnp.float32)
    pool_acc[...] += jnp.dot(onehot, h, preferred_element_type=jnp.float32)
    cnt = jnp.sum(onehot, axis=1, keepdims=True)
    cnt_acc[...] += jnp.broadcast_to(cnt, cnt_acc.shape)

    @pl.when(i == nb - 1)
    def _():
        pooled = pool_acc[...] / jnp.maximum(cnt_acc[...], 1.0)
        out_ref[...] = (
            jnp.dot(pooled, lw[...], preferred_element_type=jnp.float32)
            + lb[...])


def _tc_node_mlp(a0, a1, x, batch, nW1, nb1, nW2, nb2, nW3, nb3, lW, lb):
    BN = 400
    nblocks = N // BN
    batch3 = batch.reshape(nblocks, 1, BN)
    w1a = nW1[:NF]
    w1b = nW1[NF:]
    full = lambda shape: pl.BlockSpec(shape, lambda i: (0,) * len(shape))
    return pl.pallas_call(
        _nmlp_body,
        grid=(nblocks,),
        in_specs=[
            pl.BlockSpec((NC, BN, MSG), lambda i: (0, i, 0)),
            pl.BlockSpec((NC, BN, MSG), lambda i: (0, i, 0)),
            pl.BlockSpec((BN, NF), lambda i: (i, 0)),
            pl.BlockSpec((1, 1, BN), lambda i: (i, 0, 0)),
            full((NF, HID)),
            full((MSG, HID)),
            full((1, HID)),
            full((HID, HID)),
            full((1, HID)),
            full((HID, NH)),
            full((1, NH)),
            full((NH, NP)),
            full((1, NP)),
        ],
        out_specs=pl.BlockSpec((NG, NP), lambda i: (0, 0)),
        out_shape=jax.ShapeDtypeStruct((NG, NP), jnp.float32),
        scratch_shapes=[
            pltpu.VMEM((NG, NH), jnp.float32),
            pltpu.VMEM((NG, NH), jnp.float32),
        ],
        name="tc_node_mlp",
    )(a0, a1, x, batch3, w1a, w1b, nb1.reshape(1, HID), nW2,
      nb2.reshape(1, HID), nW3, nb3.reshape(1, NH), lW, lb.reshape(1, NP))


def kernel(x, edge_index, edge_attr, batch,
           mW1, mb1, mW2, mb2, mW3, mb3,
           nW1, nb1, nW2, nb2, nW3, nb3,
           lW, lb):
    src = edge_index[0]
    dst = edge_index[1]
    ea_t = edge_attr.T
    xj0, xi0 = _sc_gather(x, src, dst, 0)
    xj1, xi1 = _sc_gather(x, src, dst, 1)
    msg0 = _tc_edge_mlp(xi0, xj0, ea_t, 0, mW1, mb1, mW2, mb2, mW3, mb3)
    msg1 = _tc_edge_mlp(xi1, xj1, ea_t, 1, mW1, mb1, mW2, mb2, mW3, mb3)
    a0 = _sc_scatter(msg0, dst, 0)
    a1 = _sc_scatter(msg1, dst, 1)
    return _tc_node_mlp(a0, a1, x, batch, nW1, nb1, nW2, nb2, nW3, nb3,
                        lW, lb)


# speedup vs baseline: 1.9730x; 1.0422x over previous
import functools

import jax
import jax.numpy as jnp
from jax import lax
from jax.experimental import pallas as pl
from jax.experimental.pallas import tpu as pltpu
from jax.experimental.pallas import tpu_sc as plsc

N = 10000
E = 320000
NF = 128
NEF = 16
MSG = 128
HID = 300
NH = 128
NP = 2
NG = 64

NC = 2
NS = 16
NW = NC * NS

C = 80
PW = (E // NW) // 2
PW0 = 4800
PW1 = 2 * PW - PW0
ES0 = PW0 * NW
ES1 = PW1 * NW
NAGG = 10240
ROWS_PER_TILE = NAGG // NS


def _maybe_when(cond, fn):
    if isinstance(cond, bool):
        if cond:
            fn()
    else:
        pl.when(cond)(fn)


def _gather_body(goff, per_w, x_hbm, src_hbm, dst_hbm, xji_hbm,
                 idx_s, idx_d, rows_s, rows_d, sem_is, sem_id, sem_g, sem_w):
    c = lax.axis_index("c")
    s = lax.axis_index("s")
    wbase = (c * NS + s) * per_w
    nchunks = per_w // C

    def wb(p, o):
        pltpu.async_copy(rows_d[p], xji_hbm.at[pl.ds(o, C), pl.ds(0, NF)],
                         sem_w[p])
        pltpu.async_copy(rows_s[p], xji_hbm.at[pl.ds(o, C), pl.ds(NF, NF)],
                         sem_w[p])

    def stage(j, b):
        o = wbase + j * C
        g = goff + o

        def drain_prev_wb():
            pltpu.make_async_copy(rows_s[b], xji_hbm.at[pl.ds(0, C),
                                                        pl.ds(0, NF)],
                                  sem_w[b]).wait()
            pltpu.make_async_copy(rows_d[b], xji_hbm.at[pl.ds(0, C),
                                                        pl.ds(0, NF)],
                                  sem_w[b]).wait()

        _maybe_when(j >= 2, drain_prev_wb)

        ci = pltpu.async_copy(src_hbm.at[pl.ds(g, C)], idx_s[b], sem_is[b])
        cd = pltpu.async_copy(dst_hbm.at[pl.ds(g, C)], idx_d[b], sem_id[b])
        ci.wait()
        pltpu.async_copy(x_hbm.at[idx_s[b]], rows_s[b], sem_g[b])
        cd.wait()
        pltpu.async_copy(x_hbm.at[idx_d[b]], rows_d[b], sem_g[b])

        def wb_prev():
            ob = wbase + (j - 1) * C
            p = 1 - b
            pltpu.make_async_copy(x_hbm.at[idx_s[p]], rows_s[p],
                                  sem_g[p]).wait()
            pltpu.make_async_copy(x_hbm.at[idx_d[p]], rows_d[p],
                                  sem_g[p]).wait()
            wb(p, ob)

        _maybe_when(j >= 1, wb_prev)

    @pl.loop(0, nchunks // 2)
    def _(i):
        stage(2 * i, 0)
        stage(2 * i + 1, 1)

    if nchunks % 2:
        stage(nchunks - 1, 0)
    bl = (nchunks - 1) % 2
    ol = wbase + (nchunks - 1) * C
    pltpu.make_async_copy(x_hbm.at[idx_s[bl]], rows_s[bl], sem_g[bl]).wait()
    pltpu.make_async_copy(x_hbm.at[idx_d[bl]], rows_d[bl], sem_g[bl]).wait()
    pltpu.sync_copy(rows_d[bl], xji_hbm.at[pl.ds(ol, C), pl.ds(0, NF)])
    pltpu.sync_copy(rows_s[bl], xji_hbm.at[pl.ds(ol, C), pl.ds(NF, NF)])
    pltpu.make_async_copy(rows_s[1 - bl], xji_hbm.at[pl.ds(0, C),
                                                     pl.ds(0, NF)],
                          sem_w[1 - bl]).wait()
    pltpu.make_async_copy(rows_d[1 - bl], xji_hbm.at[pl.ds(0, C),
                                                     pl.ds(0, NF)],
                          sem_w[1 - bl]).wait()


def _sc_gather(x, src, dst, sl):
    goff, per_w, es = (0, PW0, ES0) if sl == 0 else (ES0, PW1, ES1)
    mesh = plsc.VectorSubcoreMesh(core_axis_name="c", subcore_axis_name="s")
    f = pl.kernel(
        functools.partial(_gather_body, goff, per_w),
        out_type=jax.ShapeDtypeStruct((es, 2 * NF), jnp.float32),
        mesh=mesh,
        scratch_types=[
            [pltpu.VMEM((C,), jnp.int32)] * 2,
            [pltpu.VMEM((C,), jnp.int32)] * 2,
            [pltpu.VMEM((C, NF), jnp.float32)] * 2,
            [pltpu.VMEM((C, NF), jnp.float32)] * 2,
            [pltpu.SemaphoreType.DMA] * 2,
            [pltpu.SemaphoreType.DMA] * 2,
            [pltpu.SemaphoreType.DMA] * 2,
            [pltpu.SemaphoreType.DMA] * 2,
        ],
        name=f"sc_gather_{sl}",
    )
    return f(x, src, dst)


def _scatter_body(goff, per_w, msg_hbm, dst_hbm, z_hbm, out_hbm,
                  idx_v, rows_v, acc_sh, sem_i, sem_m, sem_a):
    c = lax.axis_index("c")
    s = lax.axis_index("s")
    pltpu.sync_copy(z_hbm, acc_sh.at[pl.ds(s * ROWS_PER_TILE, ROWS_PER_TILE)])
    plsc.subcore_barrier()

    wbase = (c * NS + s) * per_w
    nchunks = per_w // C

    def stage(j, b):
        o = wbase + j * C
        g = goff + o

        def drain_prev_add():
            pltpu.make_async_copy(rows_v[b], acc_sh.at[idx_v[b]],
                                  sem_a[b]).wait()

        _maybe_when(j >= 2, drain_prev_add)

        ci = pltpu.async_copy(dst_hbm.at[pl.ds(g, C)], idx_v[b], sem_i[b])
        cm = pltpu.async_copy(msg_hbm.at[pl.ds(o, C)], rows_v[b], sem_m[b])
        ci.wait()
        cm.wait()
        pltpu.async_copy(rows_v[b], acc_sh.at[idx_v[b]], sem_a[b], add=True)

    @pl.loop(0, nchunks // 2)
    def _(i):
        stage(2 * i, 0)
        stage(2 * i + 1, 1)

    if nchunks % 2:
        stage(nchunks - 1, 0)
    bl = (nchunks - 1) % 2
    pltpu.make_async_copy(rows_v[bl], acc_sh.at[idx_v[bl]], sem_a[bl]).wait()
    if nchunks >= 2:
        pltpu.make_async_copy(rows_v[1 - bl], acc_sh.at[idx_v[1 - bl]],
                              sem_a[1 - bl]).wait()

    plsc.subcore_barrier()
    pltpu.sync_copy(
        acc_sh.at[pl.ds(s * ROWS_PER_TILE, ROWS_PER_TILE)],
        out_hbm.at[c].at[pl.ds(s * ROWS_PER_TILE, ROWS_PER_TILE)],
    )


def _sc_scatter(msg, dst, sl):
    goff, per_w = (0, PW0) if sl == 0 else (ES0, PW1)
    mesh = plsc.VectorSubcoreMesh(core_axis_name="c", subcore_axis_name="s")
    z = jnp.zeros((ROWS_PER_TILE, MSG), jnp.float32)
    f = pl.kernel(
        functools.partial(_scatter_body, goff, per_w),
        out_type=jax.ShapeDtypeStruct((NC, NAGG, MSG), jnp.float32),
        mesh=mesh,
        scratch_types=[
            [pltpu.VMEM((C,), jnp.int32)] * 2,
            [pltpu.VMEM((C, MSG), jnp.float32)] * 2,
            pltpu.VMEM_SHARED((NAGG, MSG), jnp.float32),
            [pltpu.SemaphoreType.DMA] * 2,
            [pltpu.SemaphoreType.DMA] * 2,
            [pltpu.SemaphoreType.DMA] * 2,
        ],
        name=f"sc_scatter_{sl}",
    )
    return f(msg, dst, z)


def _silu(v):
    return v * jax.nn.sigmoid(v)


def _bdot(a, b):
    return jnp.dot(a.astype(jnp.bfloat16), b.astype(jnp.bfloat16),
                   preferred_element_type=jnp.float32)


def _emlp_body(xji_ref, eat_ref, w1ab, w1c, b1, w2, b2, w3, b3, out_ref):
    eac = lax.dot_general(
        eat_ref[...].astype(jnp.bfloat16), w1c[...].astype(jnp.bfloat16),
        dimension_numbers=(((0,), (0,)), ((), ())),
        preferred_element_type=jnp.float32)
    h = _bdot(xji_ref[...], w1ab[...]) + eac + b1[...]
    h = _silu(h)
    h = _silu(_bdot(h, w2[...]) + b2[...])
    out_ref[...] = _bdot(h, w3[...]) + b3[...]


def _tc_edge_mlp(xji, ea_t, sl, mW1, mb1, mW2, mb2, mW3, mb3):
    BE = 1280
    es = xji.shape[0]
    grid = (es // BE,)
    ea_off = 0 if sl == 0 else ES0 // BE
    w1ab = mW1[:2 * NF]
    w1c = mW1[2 * NF:]
    full = lambda shape: pl.BlockSpec(shape, lambda i: (0,) * len(shape))
    return pl.pallas_call(
        _emlp_body,
        grid=grid,
        in_specs=[
            pl.BlockSpec((BE, 2 * NF), lambda i: (i, 0)),
            pl.BlockSpec((NEF, BE), lambda i: (0, i + ea_off)),
            full((2 * NF, HID)),
            full((NEF, HID)),
            full((1, HID)),
            full((HID, HID)),
            full((1, HID)),
            full((HID, MSG)),
            full((1, MSG)),
        ],
        out_specs=pl.BlockSpec((BE, MSG), lambda i: (i, 0)),
        out_shape=jax.ShapeDtypeStruct((es, MSG), jnp.float32),
        name=f"tc_edge_mlp_{sl}",
    )(xji, ea_t, w1ab, w1c, mb1.reshape(1, HID), mW2,
      mb2.reshape(1, HID), mW3, mb3.reshape(1, MSG))


def _nmlp_body(a0_ref, a1_ref, x_ref, batch_ref, w1a, w1b, b1, w2, b2, w3,
               b3, lw, lb, out_ref, pool_acc, cnt_acc):
    i = pl.program_id(0)
    nb = pl.num_programs(0)

    @pl.when(i == 0)
    def _():
        pool_acc[...] = jnp.zeros_like(pool_acc)
        cnt_acc[...] = jnp.zeros_like(cnt_acc)

    aggr = (a0_ref[0] + a0_ref[1]) + (a1_ref[0] + a1_ref[1])
    h = (_bdot(x_ref[...], w1a[...])
         + _bdot(aggr, w1b[...])
         + b1[...])
    h = _silu(h)
    h = _silu(_bdot(h, w2[...]) + b2[...])
    h = _bdot(h, w3[...]) + b3[...]

    ids = batch_ref[...].reshape(1, -1)
    iota = lax.broadcasted_iota(jnp.int32, (NG, ids.shape[1]), 0)
    onehot = (iota == ids).astype(jnp.float32)
    pool_acc[...] += jnp.dot(onehot, h, preferred_element_type=jnp.float32)
    cnt = jnp.sum(onehot, axis=1, keepdims=True)
    cnt_acc[...] += jnp.broadcast_to(cnt, cnt_acc.shape)

    @pl.when(i == nb - 1)
    def _():
        pooled = pool_acc[...] / jnp.maximum(cnt_acc[...], 1.0)
        out_ref[...] = (
            jnp.dot(pooled, lw[...], preferred_element_type=jnp.float32)
            + lb[...])


def _tc_node_mlp(a0, a1, x, batch, nW1, nb1, nW2, nb2, nW3, nb3, lW, lb):
    BN = 400
    nblocks = N // BN
    batch3 = batch.reshape(nblocks, 1, BN)
    w1a = nW1[:NF]
    w1b = nW1[NF:]
    full = lambda shape: pl.BlockSpec(shape, lambda i: (0,) * len(shape))
    return pl.pallas_call(
        _nmlp_body,
        grid=(nblocks,),
        in_specs=[
            pl.BlockSpec((NC, BN, MSG), lambda i: (0, i, 0)),
            pl.BlockSpec((NC, BN, MSG), lambda i: (0, i, 0)),
            pl.BlockSpec((BN, NF), lambda i: (i, 0)),
            pl.BlockSpec((1, 1, BN), lambda i: (i, 0, 0)),
            full((NF, HID)),
            full((MSG, HID)),
            full((1, HID)),
            full((HID, HID)),
            full((1, HID)),
            full((HID, NH)),
            full((1, NH)),
            full((NH, NP)),
            full((1, NP)),
        ],
        out_specs=pl.BlockSpec((NG, NP), lambda i: (0, 0)),
        out_shape=jax.ShapeDtypeStruct((NG, NP), jnp.float32),
        scratch_shapes=[
            pltpu.VMEM((NG, NH), jnp.float32),
            pltpu.VMEM((NG, NH), jnp.float32),
        ],
        name="tc_node_mlp",
    )(a0, a1, x, batch3, w1a, w1b, nb1.reshape(1, HID), nW2,
      nb2.reshape(1, HID), nW3, nb3.reshape(1, NH), lW, lb.reshape(1, NP))


def kernel(x, edge_index, edge_attr, batch,
           mW1, mb1, mW2, mb2, mW3, mb3,
           nW1, nb1, nW2, nb2, nW3, nb3,
           lW, lb):
    src = edge_index[0]
    dst = edge_index[1]
    ea_t = edge_attr.T
    xji0 = _sc_gather(x, src, dst, 0)
    xji1 = _sc_gather(x, src, dst, 1)
    msg0 = _tc_edge_mlp(xji0, ea_t, 0, mW1, mb1, mW2, mb2, mW3, mb3)
    msg1 = _tc_edge_mlp(xji1, ea_t, 1, mW1, mb1, mW2, mb2, mW3, mb3)
    a0 = _sc_scatter(msg0, dst, 0)
    a1 = _sc_scatter(msg1, dst, 1)
    return _tc_node_mlp(a0, a1, x, batch, nW1, nb1, nW2, nb2, nW3, nb3,
                        lW, lb)


# speedup vs baseline: 2.0608x; 1.0445x over previous
import functools

import jax
import jax.numpy as jnp
from jax import lax
from jax.experimental import pallas as pl
from jax.experimental.pallas import tpu as pltpu
from jax.experimental.pallas import tpu_sc as plsc

N = 10000
E = 320000
NF = 128
NEF = 16
MSG = 128
HID = 300
NH = 128
NP = 2
NG = 64

NC = 2
NS = 16
NW = NC * NS

C = 80
PWS = (3040, 4480, 2480)
GOFFS = (0, PWS[0] * NW, (PWS[0] + PWS[1]) * NW)
ESS = tuple(p * NW for p in PWS)
NSLICES = len(PWS)
NAGG = 10240
ROWS_PER_TILE = NAGG // NS


def _maybe_when(cond, fn):
    if isinstance(cond, bool):
        if cond:
            fn()
    else:
        pl.when(cond)(fn)


def _gather_body(goff, per_w, x_hbm, src_hbm, dst_hbm, xji_hbm,
                 idx_s, idx_d, rows_s, rows_d, sem_is, sem_id, sem_g, sem_w):
    c = lax.axis_index("c")
    s = lax.axis_index("s")
    wbase = (c * NS + s) * per_w
    nchunks = per_w // C

    def wb(p, o):
        pltpu.async_copy(rows_d[p], xji_hbm.at[pl.ds(o, C), pl.ds(0, NF)],
                         sem_w[p])
        pltpu.async_copy(rows_s[p], xji_hbm.at[pl.ds(o, C), pl.ds(NF, NF)],
                         sem_w[p])

    def stage(j, b):
        o = wbase + j * C
        g = goff + o

        def drain_prev_wb():
            pltpu.make_async_copy(rows_s[b], xji_hbm.at[pl.ds(0, C),
                                                        pl.ds(0, NF)],
                                  sem_w[b]).wait()
            pltpu.make_async_copy(rows_d[b], xji_hbm.at[pl.ds(0, C),
                                                        pl.ds(0, NF)],
                                  sem_w[b]).wait()

        _maybe_when(j >= 2, drain_prev_wb)

        ci = pltpu.async_copy(src_hbm.at[pl.ds(g, C)], idx_s[b], sem_is[b])
        cd = pltpu.async_copy(dst_hbm.at[pl.ds(g, C)], idx_d[b], sem_id[b])
        ci.wait()
        pltpu.async_copy(x_hbm.at[idx_s[b]], rows_s[b], sem_g[b])
        cd.wait()
        pltpu.async_copy(x_hbm.at[idx_d[b]], rows_d[b], sem_g[b])

        def wb_prev():
            ob = wbase + (j - 1) * C
            p = 1 - b
            pltpu.make_async_copy(x_hbm.at[idx_s[p]], rows_s[p],
                                  sem_g[p]).wait()
            pltpu.make_async_copy(x_hbm.at[idx_d[p]], rows_d[p],
                                  sem_g[p]).wait()
            wb(p, ob)

        _maybe_when(j >= 1, wb_prev)

    @pl.loop(0, nchunks // 2)
    def _(i):
        stage(2 * i, 0)
        stage(2 * i + 1, 1)

    if nchunks % 2:
        stage(nchunks - 1, 0)
    bl = (nchunks - 1) % 2
    ol = wbase + (nchunks - 1) * C
    pltpu.make_async_copy(x_hbm.at[idx_s[bl]], rows_s[bl], sem_g[bl]).wait()
    pltpu.make_async_copy(x_hbm.at[idx_d[bl]], rows_d[bl], sem_g[bl]).wait()
    pltpu.sync_copy(rows_d[bl], xji_hbm.at[pl.ds(ol, C), pl.ds(0, NF)])
    pltpu.sync_copy(rows_s[bl], xji_hbm.at[pl.ds(ol, C), pl.ds(NF, NF)])
    pltpu.make_async_copy(rows_s[1 - bl], xji_hbm.at[pl.ds(0, C),
                                                     pl.ds(0, NF)],
                          sem_w[1 - bl]).wait()
    pltpu.make_async_copy(rows_d[1 - bl], xji_hbm.at[pl.ds(0, C),
                                                     pl.ds(0, NF)],
                          sem_w[1 - bl]).wait()


def _sc_gather(x, src, dst, sl):
    goff, per_w, es = GOFFS[sl], PWS[sl], ESS[sl]
    mesh = plsc.VectorSubcoreMesh(core_axis_name="c", subcore_axis_name="s")
    f = pl.kernel(
        functools.partial(_gather_body, goff, per_w),
        out_type=jax.ShapeDtypeStruct((es, 2 * NF), jnp.float32),
        mesh=mesh,
        scratch_types=[
            [pltpu.VMEM((C,), jnp.int32)] * 2,
            [pltpu.VMEM((C,), jnp.int32)] * 2,
            [pltpu.VMEM((C, NF), jnp.float32)] * 2,
            [pltpu.VMEM((C, NF), jnp.float32)] * 2,
            [pltpu.SemaphoreType.DMA] * 2,
            [pltpu.SemaphoreType.DMA] * 2,
            [pltpu.SemaphoreType.DMA] * 2,
            [pltpu.SemaphoreType.DMA] * 2,
        ],
        name=f"sc_gather_{sl}",
    )
    return f(x, src, dst)


def _scatter_body(goff, per_w, msg_hbm, dst_hbm, z_hbm, out_hbm,
                  idx_v, rows_v, acc_sh, sem_i, sem_m, sem_a):
    c = lax.axis_index("c")
    s = lax.axis_index("s")
    pltpu.sync_copy(z_hbm, acc_sh.at[pl.ds(s * ROWS_PER_TILE, ROWS_PER_TILE)])
    plsc.subcore_barrier()

    wbase = (c * NS + s) * per_w
    nchunks = per_w // C

    def stage(j, b):
        o = wbase + j * C
        g = goff + o

        def drain_prev_add():
            pltpu.make_async_copy(rows_v[b], acc_sh.at[idx_v[b]],
                                  sem_a[b]).wait()

        _maybe_when(j >= 2, drain_prev_add)

        ci = pltpu.async_copy(dst_hbm.at[pl.ds(g, C)], idx_v[b], sem_i[b])
        cm = pltpu.async_copy(msg_hbm.at[pl.ds(o, C)], rows_v[b], sem_m[b])
        ci.wait()
        cm.wait()
        pltpu.async_copy(rows_v[b], acc_sh.at[idx_v[b]], sem_a[b], add=True)

    @pl.loop(0, nchunks // 2)
    def _(i):
        stage(2 * i, 0)
        stage(2 * i + 1, 1)

    if nchunks % 2:
        stage(nchunks - 1, 0)
    bl = (nchunks - 1) % 2
    pltpu.make_async_copy(rows_v[bl], acc_sh.at[idx_v[bl]], sem_a[bl]).wait()
    if nchunks >= 2:
        pltpu.make_async_copy(rows_v[1 - bl], acc_sh.at[idx_v[1 - bl]],
                              sem_a[1 - bl]).wait()

    plsc.subcore_barrier()
    pltpu.sync_copy(
        acc_sh.at[pl.ds(s * ROWS_PER_TILE, ROWS_PER_TILE)],
        out_hbm.at[c].at[pl.ds(s * ROWS_PER_TILE, ROWS_PER_TILE)],
    )


def _sc_scatter(msg, dst, sl):
    goff, per_w = GOFFS[sl], PWS[sl]
    mesh = plsc.VectorSubcoreMesh(core_axis_name="c", subcore_axis_name="s")
    z = jnp.zeros((ROWS_PER_TILE, MSG), jnp.float32)
    f = pl.kernel(
        functools.partial(_scatter_body, goff, per_w),
        out_type=jax.ShapeDtypeStruct((NC, NAGG, MSG), jnp.float32),
        mesh=mesh,
        scratch_types=[
            [pltpu.VMEM((C,), jnp.int32)] * 2,
            [pltpu.VMEM((C, MSG), jnp.float32)] * 2,
            pltpu.VMEM_SHARED((NAGG, MSG), jnp.float32),
            [pltpu.SemaphoreType.DMA] * 2,
            [pltpu.SemaphoreType.DMA] * 2,
            [pltpu.SemaphoreType.DMA] * 2,
        ],
        name=f"sc_scatter_{sl}",
    )
    return f(msg, dst, z)


def _silu(v):
    return v * jax.nn.sigmoid(v)


def _bdot(a, b):
    return jnp.dot(a.astype(jnp.bfloat16), b.astype(jnp.bfloat16),
                   preferred_element_type=jnp.float32)


def _emlp_body(xji_ref, eat_ref, w1ab, w1c, b1, w2, b2, w3, b3, out_ref):
    eac = lax.dot_general(
        eat_ref[...].astype(jnp.bfloat16), w1c[...].astype(jnp.bfloat16),
        dimension_numbers=(((0,), (0,)), ((), ())),
        preferred_element_type=jnp.float32)
    h = _bdot(xji_ref[...], w1ab[...]) + eac + b1[...]
    h = _silu(h)
    h = _silu(_bdot(h, w2[...]) + b2[...])
    out_ref[...] = _bdot(h, w3[...]) + b3[...]


def _tc_edge_mlp(xji, ea_t, sl, mW1, mb1, mW2, mb2, mW3, mb3):
    BE = 1280
    es = xji.shape[0]
    grid = (es // BE,)
    ea_off = GOFFS[sl] // BE
    w1ab = mW1[:2 * NF]
    w1c = mW1[2 * NF:]
    full = lambda shape: pl.BlockSpec(shape, lambda i: (0,) * len(shape))
    return pl.pallas_call(
        _emlp_body,
        grid=grid,
        in_specs=[
            pl.BlockSpec((BE, 2 * NF), lambda i: (i, 0)),
            pl.BlockSpec((NEF, BE), lambda i: (0, i + ea_off)),
            full((2 * NF, HID)),
            full((NEF, HID)),
            full((1, HID)),
            full((HID, HID)),
            full((1, HID)),
            full((HID, MSG)),
            full((1, MSG)),
        ],
        out_specs=pl.BlockSpec((BE, MSG), lambda i: (i, 0)),
        out_shape=jax.ShapeDtypeStruct((es, MSG), jnp.float32),
        name=f"tc_edge_mlp_{sl}",
    )(xji, ea_t, w1ab, w1c, mb1.reshape(1, HID), mW2,
      mb2.reshape(1, HID), mW3, mb3.reshape(1, MSG))


def _nmlp_body(a0_ref, a1_ref, a2_ref, x_ref, batch_ref, w1a, w1b, b1, w2,
               b2, w3, b3, lw, lb, out_ref, pool_acc, cnt_acc):
    i = pl.program_id(0)
    nb = pl.num_programs(0)

    @pl.when(i == 0)
    def _():
        pool_acc[...] = jnp.zeros_like(pool_acc)
        cnt_acc[...] = jnp.zeros_like(cnt_acc)

    aggr = ((a0_ref[0] + a0_ref[1]) + (a1_ref[0] + a1_ref[1])
            + (a2_ref[0] + a2_ref[1]))
    h = (_bdot(x_ref[...], w1a[...])
         + _bdot(aggr, w1b[...])
         + b1[...])
    h = _silu(h)
    h = _silu(_bdot(h, w2[...]) + b2[...])
    h = _bdot(h, w3[...]) + b3[...]

    ids = batch_ref[...].reshape(1, -1)
    iota = lax.broadcasted_iota(jnp.int32, (NG, ids.shape[1]), 0)
    onehot = (iota == ids).astype(jnp.float32)
    pool_acc[...] += jnp.dot(onehot, h, preferred_element_type=jnp.float32)
    cnt = jnp.sum(onehot, axis=1, keepdims=True)
    cnt_acc[...] += jnp.broadcast_to(cnt, cnt_acc.shape)

    @pl.when(i == nb - 1)
    def _():
        pooled = pool_acc[...] / jnp.maximum(cnt_acc[...], 1.0)
        out_ref[...] = (
            jnp.dot(pooled, lw[...], preferred_element_type=jnp.float32)
            + lb[...])


def _tc_node_mlp(a0, a1, a2, x, batch, nW1, nb1, nW2, nb2, nW3, nb3, lW, lb):
    BN = 400
    nblocks = N // BN
    batch3 = batch.reshape(nblocks, 1, BN)
    w1a = nW1[:NF]
    w1b = nW1[NF:]
    full = lambda shape: pl.BlockSpec(shape, lambda i: (0,) * len(shape))
    return pl.pallas_call(
        _nmlp_body,
        grid=(nblocks,),
        in_specs=[
            pl.BlockSpec((NC, BN, MSG), lambda i: (0, i, 0)),
            pl.BlockSpec((NC, BN, MSG), lambda i: (0, i, 0)),
            pl.BlockSpec((NC, BN, MSG), lambda i: (0, i, 0)),
            pl.BlockSpec((BN, NF), lambda i: (i, 0)),
            pl.BlockSpec((1, 1, BN), lambda i: (i, 0, 0)),
            full((NF, HID)),
            full((MSG, HID)),
            full((1, HID)),
            full((HID, HID)),
            full((1, HID)),
            full((HID, NH)),
            full((1, NH)),
            full((NH, NP)),
            full((1, NP)),
        ],
        out_specs=pl.BlockSpec((NG, NP), lambda i: (0, 0)),
        out_shape=jax.ShapeDtypeStruct((NG, NP), jnp.float32),
        scratch_shapes=[
            pltpu.VMEM((NG, NH), jnp.float32),
            pltpu.VMEM((NG, NH), jnp.float32),
        ],
        name="tc_node_mlp",
    )(a0, a1, a2, x, batch3, w1a, w1b, nb1.reshape(1, HID), nW2,
      nb2.reshape(1, HID), nW3, nb3.reshape(1, NH), lW, lb.reshape(1, NP))


def kernel(x, edge_index, edge_attr, batch,
           mW1, mb1, mW2, mb2, mW3, mb3,
           nW1, nb1, nW2, nb2, nW3, nb3,
           lW, lb):
    src = edge_index[0]
    dst = edge_index[1]
    ea_t = edge_attr.T
    xjis = [_sc_gather(x, src, dst, sl) for sl in range(NSLICES)]
    msgs = [_tc_edge_mlp(xjis[sl], ea_t, sl, mW1, mb1, mW2, mb2, mW3, mb3)
            for sl in range(NSLICES)]
    aggs = [_sc_scatter(msgs[sl], dst, sl) for sl in range(NSLICES)]
    return _tc_node_mlp(*aggs, x, batch, nW1, nb1, nW2, nb2, nW3, nb3,
                        lW, lb)


# speedup vs baseline: 2.0740x; 1.0064x over previous
import functools

import jax
import jax.numpy as jnp
from jax import lax
from jax.experimental import pallas as pl
from jax.experimental.pallas import tpu as pltpu
from jax.experimental.pallas import tpu_sc as plsc

N = 10000
E = 320000
NF = 128
NEF = 16
MSG = 128
HID = 300
NH = 128
NP = 2
NG = 64

NC = 2
NS = 16
NW = NC * NS

C = 80
PWS = (3040, 4480, 2480)
GOFFS = (0, PWS[0] * NW, (PWS[0] + PWS[1]) * NW)
ESS = tuple(p * NW for p in PWS)
NSLICES = len(PWS)
NAGG = 10240
ROWS_PER_TILE = NAGG // NS


def _maybe_when(cond, fn):
    if isinstance(cond, bool):
        if cond:
            fn()
    else:
        pl.when(cond)(fn)


def _gather_body(goff, per_w, x_hbm, src_hbm, dst_hbm, xji_hbm,
                 idx_s, idx_d, rows_s, rows_d, sem_is, sem_id, sem_g, sem_w):
    c = lax.axis_index("c")
    s = lax.axis_index("s")
    wbase = (c * NS + s) * per_w
    nchunks = per_w // C

    def wb(p, o):
        pltpu.async_copy(rows_d[p], xji_hbm.at[pl.ds(o, C), pl.ds(0, NF)],
                         sem_w[p])
        pltpu.async_copy(rows_s[p], xji_hbm.at[pl.ds(o, C), pl.ds(NF, NF)],
                         sem_w[p])

    def stage(j, b):
        o = wbase + j * C
        g = goff + o

        def drain_prev_wb():
            pltpu.make_async_copy(rows_s[b], xji_hbm.at[pl.ds(0, C),
                                                        pl.ds(0, NF)],
                                  sem_w[b]).wait()
            pltpu.make_async_copy(rows_d[b], xji_hbm.at[pl.ds(0, C),
                                                        pl.ds(0, NF)],
                                  sem_w[b]).wait()

        _maybe_when(j >= 2, drain_prev_wb)

        ci = pltpu.async_copy(src_hbm.at[pl.ds(g, C)], idx_s[b], sem_is[b])
        cd = pltpu.async_copy(dst_hbm.at[pl.ds(g, C)], idx_d[b], sem_id[b])
        ci.wait()
        pltpu.async_copy(x_hbm.at[idx_s[b]], rows_s[b], sem_g[b])
        cd.wait()
        pltpu.async_copy(x_hbm.at[idx_d[b]], rows_d[b], sem_g[b])

        def wb_prev():
            ob = wbase + (j - 1) * C
            p = 1 - b
            pltpu.make_async_copy(x_hbm.at[idx_s[p]], rows_s[p],
                                  sem_g[p]).wait()
            pltpu.make_async_copy(x_hbm.at[idx_d[p]], rows_d[p],
                                  sem_g[p]).wait()
            wb(p, ob)

        _maybe_when(j >= 1, wb_prev)

    @pl.loop(0, nchunks // 2)
    def _(i):
        stage(2 * i, 0)
        stage(2 * i + 1, 1)

    if nchunks % 2:
        stage(nchunks - 1, 0)
    bl = (nchunks - 1) % 2
    ol = wbase + (nchunks - 1) * C
    pltpu.make_async_copy(x_hbm.at[idx_s[bl]], rows_s[bl], sem_g[bl]).wait()
    pltpu.make_async_copy(x_hbm.at[idx_d[bl]], rows_d[bl], sem_g[bl]).wait()
    pltpu.sync_copy(rows_d[bl], xji_hbm.at[pl.ds(ol, C), pl.ds(0, NF)])
    pltpu.sync_copy(rows_s[bl], xji_hbm.at[pl.ds(ol, C), pl.ds(NF, NF)])
    pltpu.make_async_copy(rows_s[1 - bl], xji_hbm.at[pl.ds(0, C),
                                                     pl.ds(0, NF)],
                          sem_w[1 - bl]).wait()
    pltpu.make_async_copy(rows_d[1 - bl], xji_hbm.at[pl.ds(0, C),
                                                     pl.ds(0, NF)],
                          sem_w[1 - bl]).wait()


def _sc_gather(x, src, dst, sl):
    goff, per_w, es = GOFFS[sl], PWS[sl], ESS[sl]
    mesh = plsc.VectorSubcoreMesh(core_axis_name="c", subcore_axis_name="s")
    f = pl.kernel(
        functools.partial(_gather_body, goff, per_w),
        out_type=jax.ShapeDtypeStruct((es, 2 * NF), jnp.float32),
        mesh=mesh,
        scratch_types=[
            [pltpu.VMEM((C,), jnp.int32)] * 2,
            [pltpu.VMEM((C,), jnp.int32)] * 2,
            [pltpu.VMEM((C, NF), jnp.float32)] * 2,
            [pltpu.VMEM((C, NF), jnp.float32)] * 2,
            [pltpu.SemaphoreType.DMA] * 2,
            [pltpu.SemaphoreType.DMA] * 2,
            [pltpu.SemaphoreType.DMA] * 2,
            [pltpu.SemaphoreType.DMA] * 2,
        ],
        name=f"sc_gather_{sl}",
    )
    return f(x, src, dst)


def _scatter_body(goff, per_w, msg_hbm, dst_hbm, z_hbm, out_hbm,
                  idx_v, rows_v, acc_sh, sem_i, sem_m, sem_a):
    c = lax.axis_index("c")
    s = lax.axis_index("s")
    pltpu.sync_copy(z_hbm, acc_sh.at[pl.ds(s * ROWS_PER_TILE, ROWS_PER_TILE)])
    plsc.subcore_barrier()

    wbase = (c * NS + s) * per_w
    nchunks = per_w // C

    def stage(j, b):
        o = wbase + j * C
        g = goff + o

        def drain_prev_add():
            pltpu.make_async_copy(rows_v[b], acc_sh.at[idx_v[b]],
                                  sem_a[b]).wait()

        _maybe_when(j >= 2, drain_prev_add)

        ci = pltpu.async_copy(dst_hbm.at[pl.ds(g, C)], idx_v[b], sem_i[b])
        cm = pltpu.async_copy(msg_hbm.at[pl.ds(o, C)], rows_v[b], sem_m[b])
        ci.wait()
        cm.wait()
        pltpu.async_copy(rows_v[b], acc_sh.at[idx_v[b]], sem_a[b], add=True)

    @pl.loop(0, nchunks // 2)
    def _(i):
        stage(2 * i, 0)
        stage(2 * i + 1, 1)

    if nchunks % 2:
        stage(nchunks - 1, 0)
    bl = (nchunks - 1) % 2
    pltpu.make_async_copy(rows_v[bl], acc_sh.at[idx_v[bl]], sem_a[bl]).wait()
    if nchunks >= 2:
        pltpu.make_async_copy(rows_v[1 - bl], acc_sh.at[idx_v[1 - bl]],
                              sem_a[1 - bl]).wait()

    plsc.subcore_barrier()
    pltpu.sync_copy(
        acc_sh.at[pl.ds(s * ROWS_PER_TILE, ROWS_PER_TILE)],
        out_hbm.at[c].at[pl.ds(s * ROWS_PER_TILE, ROWS_PER_TILE)],
    )


def _sc_scatter(msg, dst, sl):
    goff, per_w = GOFFS[sl], PWS[sl]
    mesh = plsc.VectorSubcoreMesh(core_axis_name="c", subcore_axis_name="s")
    z = jnp.zeros((ROWS_PER_TILE, MSG), jnp.float32)
    f = pl.kernel(
        functools.partial(_scatter_body, goff, per_w),
        out_type=jax.ShapeDtypeStruct((NC, NAGG, MSG), jnp.float32),
        mesh=mesh,
        scratch_types=[
            [pltpu.VMEM((C,), jnp.int32)] * 2,
            [pltpu.VMEM((C, MSG), jnp.float32)] * 2,
            pltpu.VMEM_SHARED((NAGG, MSG), jnp.float32),
            [pltpu.SemaphoreType.DMA] * 2,
            [pltpu.SemaphoreType.DMA] * 2,
            [pltpu.SemaphoreType.DMA] * 2,
        ],
        name=f"sc_scatter_{sl}",
    )
    return f(msg, dst, z)


def _silu(v):
    return v * jax.nn.sigmoid(v)


def _silu_b(v):
    vb = v.astype(jnp.bfloat16)
    return vb * jax.nn.sigmoid(vb)


def _bdot(a, b):
    return jnp.dot(a.astype(jnp.bfloat16), b.astype(jnp.bfloat16),
                   preferred_element_type=jnp.float32)


def _emlp_body(xji_ref, eat_ref, w1ab, w1c, b1, w2, b2, w3, b3, out_ref):
    eac = lax.dot_general(
        eat_ref[...].astype(jnp.bfloat16), w1c[...].astype(jnp.bfloat16),
        dimension_numbers=(((0,), (0,)), ((), ())),
        preferred_element_type=jnp.float32)
    h = _bdot(xji_ref[...], w1ab[...]) + eac + b1[...]
    h = _silu_b(h)
    h = _silu_b(_bdot(h, w2[...]) + b2[...])
    out_ref[...] = _bdot(h, w3[...]) + b3[...]


def _tc_edge_mlp(xji, ea_t, sl, mW1, mb1, mW2, mb2, mW3, mb3):
    BE = 1280
    es = xji.shape[0]
    grid = (es // BE,)
    ea_off = GOFFS[sl] // BE
    w1ab = mW1[:2 * NF]
    w1c = mW1[2 * NF:]
    full = lambda shape: pl.BlockSpec(shape, lambda i: (0,) * len(shape))
    return pl.pallas_call(
        _emlp_body,
        grid=grid,
        in_specs=[
            pl.BlockSpec((BE, 2 * NF), lambda i: (i, 0)),
            pl.BlockSpec((NEF, BE), lambda i: (0, i + ea_off)),
            full((2 * NF, HID)),
            full((NEF, HID)),
            full((1, HID)),
            full((HID, HID)),
            full((1, HID)),
            full((HID, MSG)),
            full((1, MSG)),
        ],
        out_specs=pl.BlockSpec((BE, MSG), lambda i: (i, 0)),
        out_shape=jax.ShapeDtypeStruct((es, MSG), jnp.float32),
        name=f"tc_edge_mlp_{sl}",
    )(xji, ea_t, w1ab, w1c, mb1.reshape(1, HID), mW2,
      mb2.reshape(1, HID), mW3, mb3.reshape(1, MSG))


def _nmlp_body(a0_ref, a1_ref, a2_ref, x_ref, batch_ref, w1a, w1b, b1, w2,
               b2, w3, b3, lw, lb, out_ref, pool_acc, cnt_acc):
    i = pl.program_id(0)
    nb = pl.num_programs(0)

    @pl.when(i == 0)
    def _():
        pool_acc[...] = jnp.zeros_like(pool_acc)
        cnt_acc[...] = jnp.zeros_like(cnt_acc)

    aggr = ((a0_ref[0] + a0_ref[1]) + (a1_ref[0] + a1_ref[1])
            + (a2_ref[0] + a2_ref[1]))
    h = (_bdot(x_ref[...], w1a[...])
         + _bdot(aggr, w1b[...])
         + b1[...])
    h = _silu(h)
    h = _silu(_bdot(h, w2[...]) + b2[...])
    h = _bdot(h, w3[...]) + b3[...]

    ids = batch_ref[...].reshape(1, -1)
    iota = lax.broadcasted_iota(jnp.int32, (NG, ids.shape[1]), 0)
    onehot = (iota == ids).astype(jnp.float32)
    pool_acc[...] += jnp.dot(onehot, h, preferred_element_type=jnp.float32)
    cnt = jnp.sum(onehot, axis=1, keepdims=True)
    cnt_acc[...] += jnp.broadcast_to(cnt, cnt_acc.shape)

    @pl.when(i == nb - 1)
    def _():
        pooled = pool_acc[...] / jnp.maximum(cnt_acc[...], 1.0)
        out_ref[...] = (
            jnp.dot(pooled, lw[...], preferred_element_type=jnp.float32)
            + lb[...])


def _tc_node_mlp(a0, a1, a2, x, batch, nW1, nb1, nW2, nb2, nW3, nb3, lW, lb):
    BN = 400
    nblocks = N // BN
    batch3 = batch.reshape(nblocks, 1, BN)
    w1a = nW1[:NF]
    w1b = nW1[NF:]
    full = lambda shape: pl.BlockSpec(shape, lambda i: (0,) * len(shape))
    return pl.pallas_call(
        _nmlp_body,
        grid=(nblocks,),
        in_specs=[
            pl.BlockSpec((NC, BN, MSG), lambda i: (0, i, 0)),
            pl.BlockSpec((NC, BN, MSG), lambda i: (0, i, 0)),
            pl.BlockSpec((NC, BN, MSG), lambda i: (0, i, 0)),
            pl.BlockSpec((BN, NF), lambda i: (i, 0)),
            pl.BlockSpec((1, 1, BN), lambda i: (i, 0, 0)),
            full((NF, HID)),
            full((MSG, HID)),
            full((1, HID)),
            full((HID, HID)),
            full((1, HID)),
            full((HID, NH)),
            full((1, NH)),
            full((NH, NP)),
            full((1, NP)),
        ],
        out_specs=pl.BlockSpec((NG, NP), lambda i: (0, 0)),
        out_shape=jax.ShapeDtypeStruct((NG, NP), jnp.float32),
        scratch_shapes=[
            pltpu.VMEM((NG, NH), jnp.float32),
            pltpu.VMEM((NG, NH), jnp.float32),
        ],
        name="tc_node_mlp",
    )(a0, a1, a2, x, batch3, w1a, w1b, nb1.reshape(1, HID), nW2,
      nb2.reshape(1, HID), nW3, nb3.reshape(1, NH), lW, lb.reshape(1, NP))


def kernel(x, edge_index, edge_attr, batch,
           mW1, mb1, mW2, mb2, mW3, mb3,
           nW1, nb1, nW2, nb2, nW3, nb3,
           lW, lb):
    src = edge_index[0]
    dst = edge_index[1]
    ea_t = edge_attr.T
    xjis = [_sc_gather(x, src, dst, sl) for sl in range(NSLICES)]
    msgs = [_tc_edge_mlp(xjis[sl], ea_t, sl, mW1, mb1, mW2, mb2, mW3, mb3)
            for sl in range(NSLICES)]
    aggs = [_sc_scatter(msgs[sl], dst, sl) for sl in range(NSLICES)]
    return _tc_node_mlp(*aggs, x, batch, nW1, nb1, nW2, nb2, nW3, nb3,
                        lW, lb)


# speedup vs baseline: 2.1305x; 1.0273x over previous
import functools

import jax
import jax.numpy as jnp
from jax import lax
from jax.experimental import pallas as pl
from jax.experimental.pallas import tpu as pltpu
from jax.experimental.pallas import tpu_sc as plsc

N = 10000
E = 320000
NF = 128
NEF = 16
MSG = 128
HID = 300
NH = 128
NP = 2
NG = 64

NC = 2
NS = 16
NW = NC * NS

C = 80
PWS = (2240, 2960, 2960, 1840)
GOFFS = tuple(sum(PWS[:i]) * NW for i in range(len(PWS)))
ESS = tuple(p * NW for p in PWS)
NSLICES = len(PWS)
NAGG = 10240
ROWS_PER_TILE = NAGG // NS


def _maybe_when(cond, fn):
    if isinstance(cond, bool):
        if cond:
            fn()
    else:
        pl.when(cond)(fn)


def _gather_body(goff, per_w, x_hbm, src_hbm, dst_hbm, xji_hbm,
                 idx_s, idx_d, rows_s, rows_d, sem_is, sem_id, sem_g, sem_w):
    c = lax.axis_index("c")
    s = lax.axis_index("s")
    wbase = (c * NS + s) * per_w
    nchunks = per_w // C

    def wb(p, o):
        pltpu.async_copy(rows_d[p], xji_hbm.at[pl.ds(o, C), pl.ds(0, NF)],
                         sem_w[p])
        pltpu.async_copy(rows_s[p], xji_hbm.at[pl.ds(o, C), pl.ds(NF, NF)],
                         sem_w[p])

    def stage(j, b):
        o = wbase + j * C
        g = goff + o

        def drain_prev_wb():
            pltpu.make_async_copy(rows_s[b], xji_hbm.at[pl.ds(0, C),
                                                        pl.ds(0, NF)],
                                  sem_w[b]).wait()
            pltpu.make_async_copy(rows_d[b], xji_hbm.at[pl.ds(0, C),
                                                        pl.ds(0, NF)],
                                  sem_w[b]).wait()

        _maybe_when(j >= 2, drain_prev_wb)

        ci = pltpu.async_copy(src_hbm.at[pl.ds(g, C)], idx_s[b], sem_is[b])
        cd = pltpu.async_copy(dst_hbm.at[pl.ds(g, C)], idx_d[b], sem_id[b])
        ci.wait()
        pltpu.async_copy(x_hbm.at[idx_s[b]], rows_s[b], sem_g[b])
        cd.wait()
        pltpu.async_copy(x_hbm.at[idx_d[b]], rows_d[b], sem_g[b])

        def wb_prev():
            ob = wbase + (j - 1) * C
            p = 1 - b
            pltpu.make_async_copy(x_hbm.at[idx_s[p]], rows_s[p],
                                  sem_g[p]).wait()
            pltpu.make_async_copy(x_hbm.at[idx_d[p]], rows_d[p],
                                  sem_g[p]).wait()
            wb(p, ob)

        _maybe_when(j >= 1, wb_prev)

    @pl.loop(0, nchunks // 2)
    def _(i):
        stage(2 * i, 0)
        stage(2 * i + 1, 1)

    if nchunks % 2:
        stage(nchunks - 1, 0)
    bl = (nchunks - 1) % 2
    ol = wbase + (nchunks - 1) * C
    pltpu.make_async_copy(x_hbm.at[idx_s[bl]], rows_s[bl], sem_g[bl]).wait()
    pltpu.make_async_copy(x_hbm.at[idx_d[bl]], rows_d[bl], sem_g[bl]).wait()
    pltpu.sync_copy(rows_d[bl], xji_hbm.at[pl.ds(ol, C), pl.ds(0, NF)])
    pltpu.sync_copy(rows_s[bl], xji_hbm.at[pl.ds(ol, C), pl.ds(NF, NF)])
    pltpu.make_async_copy(rows_s[1 - bl], xji_hbm.at[pl.ds(0, C),
                                                     pl.ds(0, NF)],
                          sem_w[1 - bl]).wait()
    pltpu.make_async_copy(rows_d[1 - bl], xji_hbm.at[pl.ds(0, C),
                                                     pl.ds(0, NF)],
                          sem_w[1 - bl]).wait()


def _sc_gather(x, src, dst, sl):
    goff, per_w, es = GOFFS[sl], PWS[sl], ESS[sl]
    mesh = plsc.VectorSubcoreMesh(core_axis_name="c", subcore_axis_name="s")
    f = pl.kernel(
        functools.partial(_gather_body, goff, per_w),
        out_type=jax.ShapeDtypeStruct((es, 2 * NF), jnp.float32),
        mesh=mesh,
        scratch_types=[
            [pltpu.VMEM((C,), jnp.int32)] * 2,
            [pltpu.VMEM((C,), jnp.int32)] * 2,
            [pltpu.VMEM((C, NF), jnp.float32)] * 2,
            [pltpu.VMEM((C, NF), jnp.float32)] * 2,
            [pltpu.SemaphoreType.DMA] * 2,
            [pltpu.SemaphoreType.DMA] * 2,
            [pltpu.SemaphoreType.DMA] * 2,
            [pltpu.SemaphoreType.DMA] * 2,
        ],
        name=f"sc_gather_{sl}",
    )
    return f(x, src, dst)


def _scatter_body(goff, per_w, msg_hbm, dst_hbm, z_hbm, out_hbm,
                  idx_v, rows_v, acc_sh, sem_i, sem_m, sem_a):
    c = lax.axis_index("c")
    s = lax.axis_index("s")
    pltpu.sync_copy(z_hbm, acc_sh.at[pl.ds(s * ROWS_PER_TILE, ROWS_PER_TILE)])
    plsc.subcore_barrier()

    wbase = (c * NS + s) * per_w
    nchunks = per_w // C

    def stage(j, b):
        o = wbase + j * C
        g = goff + o

        def drain_prev_add():
            pltpu.make_async_copy(rows_v[b], acc_sh.at[idx_v[b]],
                                  sem_a[b]).wait()

        _maybe_when(j >= 2, drain_prev_add)

        ci = pltpu.async_copy(dst_hbm.at[pl.ds(g, C)], idx_v[b], sem_i[b])
        cm = pltpu.async_copy(msg_hbm.at[pl.ds(o, C)], rows_v[b], sem_m[b])
        ci.wait()
        cm.wait()
        pltpu.async_copy(rows_v[b], acc_sh.at[idx_v[b]], sem_a[b], add=True)

    @pl.loop(0, nchunks // 2)
    def _(i):
        stage(2 * i, 0)
        stage(2 * i + 1, 1)

    if nchunks % 2:
        stage(nchunks - 1, 0)
    bl = (nchunks - 1) % 2
    pltpu.make_async_copy(rows_v[bl], acc_sh.at[idx_v[bl]], sem_a[bl]).wait()
    if nchunks >= 2:
        pltpu.make_async_copy(rows_v[1 - bl], acc_sh.at[idx_v[1 - bl]],
                              sem_a[1 - bl]).wait()

    plsc.subcore_barrier()
    pltpu.sync_copy(
        acc_sh.at[pl.ds(s * ROWS_PER_TILE, ROWS_PER_TILE)],
        out_hbm.at[c].at[pl.ds(s * ROWS_PER_TILE, ROWS_PER_TILE)],
    )


def _sc_scatter(msg, dst, sl):
    goff, per_w = GOFFS[sl], PWS[sl]
    mesh = plsc.VectorSubcoreMesh(core_axis_name="c", subcore_axis_name="s")
    z = jnp.zeros((ROWS_PER_TILE, MSG), jnp.float32)
    f = pl.kernel(
        functools.partial(_scatter_body, goff, per_w),
        out_type=jax.ShapeDtypeStruct((NC, NAGG, MSG), jnp.float32),
        mesh=mesh,
        scratch_types=[
            [pltpu.VMEM((C,), jnp.int32)] * 2,
            [pltpu.VMEM((C, MSG), jnp.float32)] * 2,
            pltpu.VMEM_SHARED((NAGG, MSG), jnp.float32),
            [pltpu.SemaphoreType.DMA] * 2,
            [pltpu.SemaphoreType.DMA] * 2,
            [pltpu.SemaphoreType.DMA] * 2,
        ],
        name=f"sc_scatter_{sl}",
    )
    return f(msg, dst, z)


def _silu(v):
    return v * jax.nn.sigmoid(v)


def _silu_b(v):
    vb = v.astype(jnp.bfloat16)
    return vb * jax.nn.sigmoid(vb)


def _bdot(a, b):
    return jnp.dot(a.astype(jnp.bfloat16), b.astype(jnp.bfloat16),
                   preferred_element_type=jnp.float32)


def _emlp_body(xji_ref, eat_ref, w1ab, w1c, b1, w2, b2, w3, b3, out_ref):
    eac = lax.dot_general(
        eat_ref[...].astype(jnp.bfloat16), w1c[...].astype(jnp.bfloat16),
        dimension_numbers=(((0,), (0,)), ((), ())),
        preferred_element_type=jnp.float32)
    h = _bdot(xji_ref[...], w1ab[...]) + eac + b1[...]
    h = _silu_b(h)
    h = _silu_b(_bdot(h, w2[...]) + b2[...])
    out_ref[...] = _bdot(h, w3[...]) + b3[...]


def _tc_edge_mlp(xji, ea_t, sl, mW1, mb1, mW2, mb2, mW3, mb3):
    BE = 1280
    es = xji.shape[0]
    grid = (es // BE,)
    ea_off = GOFFS[sl] // BE
    w1ab = mW1[:2 * NF]
    w1c = mW1[2 * NF:]
    full = lambda shape: pl.BlockSpec(shape, lambda i: (0,) * len(shape))
    return pl.pallas_call(
        _emlp_body,
        grid=grid,
        in_specs=[
            pl.BlockSpec((BE, 2 * NF), lambda i: (i, 0)),
            pl.BlockSpec((NEF, BE), lambda i: (0, i + ea_off)),
            full((2 * NF, HID)),
            full((NEF, HID)),
            full((1, HID)),
            full((HID, HID)),
            full((1, HID)),
            full((HID, MSG)),
            full((1, MSG)),
        ],
        out_specs=pl.BlockSpec((BE, MSG), lambda i: (i, 0)),
        out_shape=jax.ShapeDtypeStruct((es, MSG), jnp.float32),
        name=f"tc_edge_mlp_{sl}",
    )(xji, ea_t, w1ab, w1c, mb1.reshape(1, HID), mW2,
      mb2.reshape(1, HID), mW3, mb3.reshape(1, MSG))


def _nmlp_body(a0_ref, a1_ref, a2_ref, a3_ref, x_ref, batch_ref, w1a, w1b,
               b1, w2, b2, w3, b3, lw, lb, out_ref, pool_acc, cnt_acc):
    i = pl.program_id(0)
    nb = pl.num_programs(0)

    @pl.when(i == 0)
    def _():
        pool_acc[...] = jnp.zeros_like(pool_acc)
        cnt_acc[...] = jnp.zeros_like(cnt_acc)

    aggr = ((a0_ref[0] + a0_ref[1]) + (a1_ref[0] + a1_ref[1])
            + (a2_ref[0] + a2_ref[1]) + (a3_ref[0] + a3_ref[1]))
    h = (_bdot(x_ref[...], w1a[...])
         + _bdot(aggr, w1b[...])
         + b1[...])
    h = _silu(h)
    h = _silu(_bdot(h, w2[...]) + b2[...])
    h = _bdot(h, w3[...]) + b3[...]

    ids = batch_ref[...].reshape(1, -1)
    iota = lax.broadcasted_iota(jnp.int32, (NG, ids.shape[1]), 0)
    onehot = (iota == ids).astype(jnp.float32)
    pool_acc[...] += jnp.dot(onehot, h, preferred_element_type=jnp.float32)
    cnt = jnp.sum(onehot, axis=1, keepdims=True)
    cnt_acc[...] += jnp.broadcast_to(cnt, cnt_acc.shape)

    @pl.when(i == nb - 1)
    def _():
        pooled = pool_acc[...] / jnp.maximum(cnt_acc[...], 1.0)
        out_ref[...] = (
            jnp.dot(pooled, lw[...], preferred_element_type=jnp.float32)
            + lb[...])


def _tc_node_mlp(a0, a1, a2, a3, x, batch, nW1, nb1, nW2, nb2, nW3, nb3,
                 lW, lb):
    BN = 400
    nblocks = N // BN
    batch3 = batch.reshape(nblocks, 1, BN)
    w1a = nW1[:NF]
    w1b = nW1[NF:]
    full = lambda shape: pl.BlockSpec(shape, lambda i: (0,) * len(shape))
    return pl.pallas_call(
        _nmlp_body,
        grid=(nblocks,),
        in_specs=[
            pl.BlockSpec((NC, BN, MSG), lambda i: (0, i, 0)),
            pl.BlockSpec((NC, BN, MSG), lambda i: (0, i, 0)),
            pl.BlockSpec((NC, BN, MSG), lambda i: (0, i, 0)),
            pl.BlockSpec((NC, BN, MSG), lambda i: (0, i, 0)),
            pl.BlockSpec((BN, NF), lambda i: (i, 0)),
            pl.BlockSpec((1, 1, BN), lambda i: (i, 0, 0)),
            full((NF, HID)),
            full((MSG, HID)),
            full((1, HID)),
            full((HID, HID)),
            full((1, HID)),
            full((HID, NH)),
            full((1, NH)),
            full((NH, NP)),
            full((1, NP)),
        ],
        out_specs=pl.BlockSpec((NG, NP), lambda i: (0, 0)),
        out_shape=jax.ShapeDtypeStruct((NG, NP), jnp.float32),
        scratch_shapes=[
            pltpu.VMEM((NG, NH), jnp.float32),
            pltpu.VMEM((NG, NH), jnp.float32),
        ],
        name="tc_node_mlp",
    )(a0, a1, a2, a3, x, batch3, w1a, w1b, nb1.reshape(1, HID), nW2,
      nb2.reshape(1, HID), nW3, nb3.reshape(1, NH), lW, lb.reshape(1, NP))


def kernel(x, edge_index, edge_attr, batch,
           mW1, mb1, mW2, mb2, mW3, mb3,
           nW1, nb1, nW2, nb2, nW3, nb3,
           lW, lb):
    src = edge_index[0]
    dst = edge_index[1]
    ea_t = edge_attr.T
    xjis = [_sc_gather(x, src, dst, sl) for sl in range(NSLICES)]
    msgs = [_tc_edge_mlp(xjis[sl], ea_t, sl, mW1, mb1, mW2, mb2, mW3, mb3)
            for sl in range(NSLICES)]
    aggs = [_sc_scatter(msgs[sl], dst, sl) for sl in range(NSLICES)]
    return _tc_node_mlp(*aggs, x, batch, nW1, nb1, nW2, nb2, nW3, nb3,
                        lW, lb)
